# Initial kernel scaffold; baseline (speedup 1.0000x reference)
import functools
import jax
import jax.numpy as jnp
from jax import lax
from jax.experimental import pallas as pl
from jax.experimental.pallas import tpu as pltpu
from jax.experimental.pallas import tpu_sc as plsc

N = 10000
IN_DIM = 256
OUT_DIM = 256
HEADS = 4
C = OUT_DIM // HEADS
G = 64
EPS = 1e-5

NB = 79
NP = NB * 128
ROWS_PER_TILE = NP // 16
DUMMY = N

E_TOT = 160000 + N
CHUNK = 128
CHUNKS_PER_TILE = 84
EP = 16 * CHUNKS_PER_TILE * CHUNK


def _s1_body(x_ref, wl_ref, bl_ref, wr_ref, br_ref, ws_ref, bs_ref,
             xl_ref, xr_ref, sk_ref):
    xb = x_ref[...]
    xl = jnp.dot(xb, wl_ref[...], preferred_element_type=jnp.float32) + bl_ref[...]
    xr = jnp.dot(xb, wr_ref[...], preferred_element_type=jnp.float32) + br_ref[...]
    xl_ref[0] = xl[:, :128]
    xl_ref[1] = xl[:, 128:]
    xr_ref[0] = xr[:, :128]
    xr_ref[1] = xr[:, 128:]
    sk_ref[...] = jnp.dot(xb, ws_ref[...], preferred_element_type=jnp.float32) + bs_ref[...]


def _stage1(xp, W_l, b_l, W_r, b_r, W_skip, b_skip):
    full = lambda s: pl.BlockSpec(s, lambda i: (0,) * len(s))
    return pl.pallas_call(
        _s1_body,
        grid=(NB,),
        in_specs=[
            pl.BlockSpec((128, IN_DIM), lambda i: (i, 0)),
            full((IN_DIM, OUT_DIM)), full((1, OUT_DIM)),
            full((IN_DIM, OUT_DIM)), full((1, OUT_DIM)),
            full((IN_DIM, OUT_DIM)), full((1, OUT_DIM)),
        ],
        out_specs=[
            pl.BlockSpec((2, 128, 128), lambda i: (0, i, 0)),
            pl.BlockSpec((2, 128, 128), lambda i: (0, i, 0)),
            pl.BlockSpec((128, OUT_DIM), lambda i: (i, 0)),
        ],
        out_shape=[
            jax.ShapeDtypeStruct((2, NP, 128), jnp.float32),
            jax.ShapeDtypeStruct((2, NP, 128), jnp.float32),
            jax.ShapeDtypeStruct((NP, OUT_DIM), jnp.float32),
        ],
    )(xp, W_l, b_l.reshape(1, -1), W_r, b_r.reshape(1, -1),
      W_skip, b_skip.reshape(1, -1))


def _sc_body(src_hbm, dstg_hbm, dsts_hbm, xl_hbm, xr_hbm, att_hbm,
             out_hbm, den_hbm,
             srcv, dgv, dsv, sgv, dgov, attv,
             xlrows, xrrows, outrows, denrows,
             u_sh, den_sh, sem1, sem2):
    c = lax.axis_index("c")
    s = lax.axis_index("s")
    coff = c * NP
    rbase = s * ROWS_PER_TILE

    def zrow(e, carry):
        for k in range(8):
            outrows[e, pl.ds(16 * k, 16)] = jnp.zeros((16,), jnp.float32)
        denrows[e] = jnp.zeros((16,), jnp.float32)
        return carry
    lax.fori_loop(0, CHUNK, zrow, 0)

    for q in range(4):
        pltpu.sync_copy(outrows, u_sh.at[pl.ds(rbase + q * 128, 128)])
        pltpu.sync_copy(denrows, den_sh.at[pl.ds(rbase + q * 128, 128)])
    pltpu.sync_copy(outrows.at[pl.ds(0, ROWS_PER_TILE - 512)],
                    u_sh.at[pl.ds(rbase + 512, ROWS_PER_TILE - 512)])
    pltpu.sync_copy(denrows.at[pl.ds(0, ROWS_PER_TILE - 512)],
                    den_sh.at[pl.ds(rbase + 512, ROWS_PER_TILE - 512)])
    plsc.subcore_barrier()

    pltpu.sync_copy(att_hbm.at[c], attv)
    ebase = s * (CHUNKS_PER_TILE * CHUNK)

    def chunk_body(j, carry):
        eb = ebase + j * CHUNK
        pltpu.sync_copy(src_hbm.at[pl.ds(eb, CHUNK)], srcv)
        pltpu.sync_copy(dstg_hbm.at[pl.ds(eb, CHUNK)], dgv)
        pltpu.sync_copy(dsts_hbm.at[pl.ds(eb, CHUNK)], dsv)
        for k in range(8):
            sgv[pl.ds(16 * k, 16)] = srcv[pl.ds(16 * k, 16)] + coff
            dgov[pl.ds(16 * k, 16)] = dgv[pl.ds(16 * k, 16)] + coff
        cp1 = pltpu.async_copy(xl_hbm.at[sgv], xlrows, sem1)
        cp2 = pltpu.async_copy(xr_hbm.at[dgov], xrrows, sem2)
        cp1.wait()
        cp2.wait()

        def edge_body(e, carry2):
            pvs = []
            for h in range(2):
                acc = jnp.zeros((16,), jnp.float32)
                for k in range(4):
                    off = h * 64 + 16 * k
                    t = xlrows[e, pl.ds(off, 16)] + xrrows[e, pl.ds(off, 16)]
                    t = jnp.maximum(t, 0.0) + 0.2 * jnp.minimum(t, 0.0)
                    acc = acc + t * attv[pl.ds(off, 16)]
                pv = jnp.exp(jnp.broadcast_to(jnp.sum(acc), (16,)))
                for k in range(4):
                    off = h * 64 + 16 * k
                    outrows[e, pl.ds(off, 16)] = xlrows[e, pl.ds(off, 16)] * pv
                pvs.append(pv)
            lane = lax.iota(jnp.int32, 16)
            denrows[e] = jnp.where(lane == 0, pvs[0],
                                   jnp.where(lane == 1, pvs[1], 0.0))
            return carry2
        lax.fori_loop(0, CHUNK, edge_body, 0)

        pltpu.sync_copy(outrows, u_sh.at[dsv], add=True)
        pltpu.sync_copy(denrows, den_sh.at[dsv], add=True)
        return carry
    lax.fori_loop(0, CHUNKS_PER_TILE, chunk_body, 0)
    plsc.subcore_barrier()

    pltpu.sync_copy(u_sh.at[pl.ds(rbase, ROWS_PER_TILE)],
                    out_hbm.at[pl.ds(coff + rbase, ROWS_PER_TILE)])
    pltpu.sync_copy(den_sh.at[pl.ds(rbase, ROWS_PER_TILE)],
                    den_hbm.at[pl.ds(coff + rbase, ROWS_PER_TILE)])


def _stage2(src, dstg, dsts, XL, XR, ATT):
    mesh = plsc.VectorSubcoreMesh(core_axis_name="c", subcore_axis_name="s")
    f = pl.kernel(
        _sc_body,
        out_type=(
            jax.ShapeDtypeStruct((2 * NP, 128), jnp.float32),
            jax.ShapeDtypeStruct((2 * NP, 16), jnp.float32),
        ),
        mesh=mesh,
        scratch_types=[
            pltpu.VMEM((CHUNK,), jnp.int32),
            pltpu.VMEM((CHUNK,), jnp.int32),
            pltpu.VMEM((CHUNK,), jnp.int32),
            pltpu.VMEM((CHUNK,), jnp.int32),
            pltpu.VMEM((CHUNK,), jnp.int32),
            pltpu.VMEM((128,), jnp.float32),
            pltpu.VMEM((CHUNK, 128), jnp.float32),
            pltpu.VMEM((CHUNK, 128), jnp.float32),
            pltpu.VMEM((CHUNK, 128), jnp.float32),
            pltpu.VMEM((CHUNK, 16), jnp.float32),
            pltpu.VMEM_SHARED((NP, 128), jnp.float32),
            pltpu.VMEM_SHARED((NP, 16), jnp.float32),
            pltpu.SemaphoreType.DMA,
            pltpu.SemaphoreType.DMA,
        ],
    )
    return f(src, dstg, dsts, XL, XR, ATT)


def _s3_body(o0_ref, o1_ref, d0_ref, d1_ref, sk_ref, batch_ref, bias_ref,
             g1w_ref, g1b_ref, g1a_ref, g2w_ref, g2b_ref, g2a_ref,
             y_ref, s1m, s2m, s1s, s2s, cntm):
    p = pl.program_id(0)
    o0 = o0_ref[...]
    o1 = o1_ref[...]
    d0 = d0_ref[...]
    d1 = d1_ref[...]
    den = jnp.concatenate(
        [jnp.broadcast_to(d0[:, 0:1], (128, 64)),
         jnp.broadcast_to(d0[:, 1:2], (128, 64)),
         jnp.broadcast_to(d1[:, 0:1], (128, 64)),
         jnp.broadcast_to(d1[:, 1:2], (128, 64))], axis=1)
    den = jnp.maximum(den, 1e-30)
    x_main = jnp.concatenate([o0, o1], axis=1) / den + bias_ref[...]
    x_skip = sk_ref[...]

    gids = lax.broadcasted_iota(jnp.int32, (128, G), 1)
    oh = jnp.where(batch_ref[...] == gids, 1.0, 0.0)

    @pl.when(p == 0)
    def _accum():
        @pl.when(pl.program_id(1) == 0)
        def _init():
            s1m[...] = jnp.zeros_like(s1m)
            s2m[...] = jnp.zeros_like(s2m)
            s1s[...] = jnp.zeros_like(s1s)
            s2s[...] = jnp.zeros_like(s2s)
            cntm[...] = jnp.zeros_like(cntm)
        dn = (((0,), (0,)), ((), ()))
        s1m[...] += lax.dot_general(oh, x_main, dn, preferred_element_type=jnp.float32)
        s2m[...] += lax.dot_general(oh, x_main * x_main, dn, preferred_element_type=jnp.float32)
        s1s[...] += lax.dot_general(oh, x_skip, dn, preferred_element_type=jnp.float32)
        s2s[...] += lax.dot_general(oh, x_skip * x_skip, dn, preferred_element_type=jnp.float32)
        cntm[...] += lax.dot_general(oh, jnp.ones((128, OUT_DIM), jnp.float32), dn,
                                     preferred_element_type=jnp.float32)

    @pl.when(p == 1)
    def _apply():
        cnt = jnp.maximum(cntm[...], 1.0)

        def norm(h, S1, S2, w, b, a):
            mean = S1 / cnt
            var = jnp.maximum(S2 / cnt - (2.0 * a - a * a) * mean * mean, 0.0)
            std = jnp.sqrt(var + EPS)
            gm = jnp.dot(oh, mean, preferred_element_type=jnp.float32)
            gs = jnp.dot(oh, std, preferred_element_type=jnp.float32)
            return w * (h - a * gm) / gs + b

        ym = norm(x_main, s1m[...], s2m[...], g1w_ref[...], g1b_ref[...], g1a_ref[...])
        ys = norm(x_skip, s1s[...], s2s[...], g2w_ref[...], g2b_ref[...], g2a_ref[...])
        z = ym + ys
        y_ref[...] = jnp.where(z > 0, z, jnp.exp(jnp.minimum(z, 0.0)) - 1.0)


def _stage3(OUT, DEN, SK, batchp, bias, g1w, g1b, g1a, g2w, g2b, g2a):
    vec = lambda: pl.BlockSpec((1, OUT_DIM), lambda p, i: (0, 0))
    return pl.pallas_call(
        _s3_body,
        grid=(2, NB),
        in_specs=[
            pl.BlockSpec((128, 128), lambda p, i: (i, 0)),
            pl.BlockSpec((128, 128), lambda p, i: (NB + i, 0)),
            pl.BlockSpec((128, 16), lambda p, i: (i, 0)),
            pl.BlockSpec((128, 16), lambda p, i: (NB + i, 0)),
            pl.BlockSpec((128, OUT_DIM), lambda p, i: (i, 0)),
            pl.BlockSpec((128, 1), lambda p, i: (i, 0)),
            vec(), vec(), vec(), vec(), vec(), vec(), vec(),
        ],
        out_specs=pl.BlockSpec((128, OUT_DIM), lambda p, i: (i, 0)),
        out_shape=jax.ShapeDtypeStruct((NP, OUT_DIM), jnp.float32),
        scratch_shapes=[pltpu.VMEM((G, OUT_DIM), jnp.float32)] * 5,
    )(OUT, OUT, DEN, DEN, SK, batchp,
      bias.reshape(1, -1), g1w.reshape(1, -1), g1b.reshape(1, -1),
      g1a.reshape(1, -1), g2w.reshape(1, -1), g2b.reshape(1, -1),
      g2a.reshape(1, -1))


def kernel(x, edge_index, batch, W_l, b_l, W_r, b_r, att, bias,
           W_skip, b_skip, gn1_w, gn1_b, gn1_a, gn2_w, gn2_b, gn2_a):
    xp = jnp.pad(x, ((0, NP - N), (0, 0)))
    batchp = jnp.pad(batch, (0, NP - N), constant_values=G).reshape(NP, 1)

    loop = jnp.arange(N, dtype=jnp.int32)
    src = jnp.concatenate([edge_index[0], loop])
    dst = jnp.concatenate([edge_index[1], loop])
    srcp = jnp.pad(src, (0, EP - E_TOT))
    dstg = jnp.pad(dst, (0, EP - E_TOT))
    dsts = jnp.pad(dst, (0, EP - E_TOT), constant_values=DUMMY)
    ATT = att.reshape(2, 128)

    XL3, XR3, SK = _stage1(xp, W_l, b_l, W_r, b_r, W_skip, b_skip)
    OUT, DEN = _stage2(srcp, dstg, dsts,
                       XL3.reshape(2 * NP, 128), XR3.reshape(2 * NP, 128), ATT)
    y = _stage3(OUT, DEN, SK, batchp, bias,
                gn1_w, gn1_b, gn1_a, gn2_w, gn2_b, gn2_a)
    return y[:N]


# speedup vs baseline: 13.3162x; 13.3162x over previous
import functools
import jax
import jax.numpy as jnp
from jax import lax
from jax.experimental import pallas as pl
from jax.experimental.pallas import tpu as pltpu
from jax.experimental.pallas import tpu_sc as plsc

N = 10000
IN_DIM = 256
OUT_DIM = 256
HEADS = 4
C = OUT_DIM // HEADS
G = 64
EPS = 1e-5

NB = 79
NP = NB * 128
ROWS_PER_TILE = NP // 16
DUMMY = N

DROWS_PER_TILE = 80
NDEN = 16 * DROWS_PER_TILE

E_TOT = 160000 + N
CHUNK = 64
CHUNKS_PER_TILE = 168
EP = 16 * CHUNKS_PER_TILE * CHUNK


def _s1_body(x_ref, wl_ref, bl_ref, wr_ref, br_ref, ws_ref, bs_ref,
             xl_ref, xr_ref, sk_ref):
    xb = x_ref[...]
    xl = jnp.dot(xb, wl_ref[...], preferred_element_type=jnp.float32) + bl_ref[...]
    xr = jnp.dot(xb, wr_ref[...], preferred_element_type=jnp.float32) + br_ref[...]
    xl_ref[0] = xl[:, :128]
    xl_ref[1] = xl[:, 128:]
    xr_ref[0] = xr[:, :128]
    xr_ref[1] = xr[:, 128:]
    sk_ref[...] = jnp.dot(xb, ws_ref[...], preferred_element_type=jnp.float32) + bs_ref[...]


def _stage1(xp, W_l, b_l, W_r, b_r, W_skip, b_skip):
    full = lambda s: pl.BlockSpec(s, lambda i: (0,) * len(s))
    return pl.pallas_call(
        _s1_body,
        grid=(NB,),
        in_specs=[
            pl.BlockSpec((128, IN_DIM), lambda i: (i, 0)),
            full((IN_DIM, OUT_DIM)), full((1, OUT_DIM)),
            full((IN_DIM, OUT_DIM)), full((1, OUT_DIM)),
            full((IN_DIM, OUT_DIM)), full((1, OUT_DIM)),
        ],
        out_specs=[
            pl.BlockSpec((2, 128, 128), lambda i: (0, i, 0)),
            pl.BlockSpec((2, 128, 128), lambda i: (0, i, 0)),
            pl.BlockSpec((128, OUT_DIM), lambda i: (i, 0)),
        ],
        out_shape=[
            jax.ShapeDtypeStruct((2, NP, 128), jnp.float32),
            jax.ShapeDtypeStruct((2, NP, 128), jnp.float32),
            jax.ShapeDtypeStruct((NP, OUT_DIM), jnp.float32),
        ],
    )(xp, W_l, b_l.reshape(1, -1), W_r, b_r.reshape(1, -1),
      W_skip, b_skip.reshape(1, -1))


def _sc_body(src_hbm, dstg_hbm, dstp_hbm, dstm_hbm, xl_hbm, xr_hbm, att_hbm,
             out_hbm, den_hbm,
             srcv, dgv, dsv, dmv, sgv, dgov, attv,
             xlrows, xrrows, denrows,
             u_sh, den_sh, sem1, sem2):
    c = lax.axis_index("c")
    s = lax.axis_index("s")
    coff = c * NP
    rbase = s * ROWS_PER_TILE
    dbase = s * DROWS_PER_TILE

    def zrow(e, carry):
        for k in range(8):
            xlrows[e, pl.ds(16 * k, 16)] = jnp.zeros((16,), jnp.float32)
            denrows[e, pl.ds(16 * k, 16)] = jnp.zeros((16,), jnp.float32)
        return carry
    lax.fori_loop(0, CHUNK, zrow, 0)

    for q in range(9):
        pltpu.sync_copy(xlrows, u_sh.at[pl.ds(rbase + q * CHUNK, CHUNK)])
    rem = ROWS_PER_TILE - 9 * CHUNK
    pltpu.sync_copy(xlrows.at[pl.ds(0, rem)],
                    u_sh.at[pl.ds(rbase + 9 * CHUNK, rem)])
    pltpu.sync_copy(denrows, den_sh.at[pl.ds(dbase, CHUNK)])
    pltpu.sync_copy(denrows.at[pl.ds(0, DROWS_PER_TILE - CHUNK)],
                    den_sh.at[pl.ds(dbase + CHUNK, DROWS_PER_TILE - CHUNK)])

    plsc.subcore_barrier()

    pltpu.sync_copy(att_hbm.at[c], attv)
    ebase = s * (CHUNKS_PER_TILE * CHUNK)

    def chunk_body(j, carry):
        eb = ebase + j * CHUNK
        pltpu.sync_copy(src_hbm.at[pl.ds(eb, CHUNK)], srcv)
        pltpu.sync_copy(dstg_hbm.at[pl.ds(eb, CHUNK)], dgv)
        pltpu.sync_copy(dstp_hbm.at[pl.ds(eb, CHUNK)], dsv)
        pltpu.sync_copy(dstm_hbm.at[pl.ds(eb, CHUNK)], dmv)
        for k in range(CHUNK // 16):
            sgv[pl.ds(16 * k, 16)] = srcv[pl.ds(16 * k, 16)] + coff
            dgov[pl.ds(16 * k, 16)] = dgv[pl.ds(16 * k, 16)] + coff
        cp1 = pltpu.async_copy(xl_hbm.at[sgv], xlrows, sem1)
        cp2 = pltpu.async_copy(xr_hbm.at[dgov], xrrows, sem2)
        cp1.wait()
        cp2.wait()

        def edge_body(e, carry2):
            pvs = []
            for h in range(2):
                acc = jnp.zeros((16,), jnp.float32)
                for k in range(4):
                    off = h * 64 + 16 * k
                    t = xlrows[e, pl.ds(off, 16)] + xrrows[e, pl.ds(off, 16)]
                    t = jnp.maximum(t, 0.0) + 0.2 * jnp.minimum(t, 0.0)
                    acc = acc + t * attv[pl.ds(off, 16)]
                pv = jnp.exp(jnp.broadcast_to(jnp.sum(acc), (16,)))
                for k in range(4):
                    off = h * 64 + 16 * k
                    xlrows[e, pl.ds(off, 16)] = xlrows[e, pl.ds(off, 16)] * pv
                pvs.append(pv)
            lane = lax.iota(jnp.int32, 16)
            pv01 = jnp.where(lane == 0, pvs[0],
                             jnp.where(lane == 1, pvs[1], 0.0))
            mv = plsc.load_gather(dmv, [jnp.broadcast_to(e, (16,)).astype(jnp.int32)])
            for k in range(8):
                denrows[e, pl.ds(16 * k, 16)] = jnp.where(
                    mv == k, pv01, jnp.zeros((16,), jnp.float32))
            return carry2
        lax.fori_loop(0, CHUNK, edge_body, 0)

        pltpu.sync_copy(xlrows, u_sh.at[dgv], add=True)
        pltpu.sync_copy(denrows, den_sh.at[dsv], add=True)
        return carry
    lax.fori_loop(0, CHUNKS_PER_TILE, chunk_body, 0)
    plsc.subcore_barrier()

    for q in range(9):
        pltpu.sync_copy(u_sh.at[pl.ds(rbase + q * CHUNK, CHUNK)], xlrows)
        pltpu.sync_copy(xlrows, out_hbm.at[pl.ds(coff + rbase + q * CHUNK, CHUNK)])
    rem2 = ROWS_PER_TILE - 9 * CHUNK
    pltpu.sync_copy(u_sh.at[pl.ds(rbase + 9 * CHUNK, rem2)], xlrows.at[pl.ds(0, rem2)])
    pltpu.sync_copy(xlrows.at[pl.ds(0, rem2)],
                    out_hbm.at[pl.ds(coff + rbase + 9 * CHUNK, rem2)])
    pltpu.sync_copy(den_sh.at[pl.ds(dbase, CHUNK)], denrows)
    pltpu.sync_copy(denrows, den_hbm.at[pl.ds(c * NDEN + dbase, CHUNK)])
    drem = DROWS_PER_TILE - CHUNK
    pltpu.sync_copy(den_sh.at[pl.ds(dbase + CHUNK, drem)], denrows.at[pl.ds(0, drem)])
    pltpu.sync_copy(denrows.at[pl.ds(0, drem)],
                    den_hbm.at[pl.ds(c * NDEN + dbase + CHUNK, drem)])


def _stage2(src, dstg, dstp, dstm, XL, XR, ATT):
    mesh = plsc.VectorSubcoreMesh(core_axis_name="c", subcore_axis_name="s")
    f = pl.kernel(
        _sc_body,
        out_type=(
            jax.ShapeDtypeStruct((2 * NP, 128), jnp.float32),
            jax.ShapeDtypeStruct((2 * NDEN, 128), jnp.float32),
        ),
        mesh=mesh,
        compiler_params=pltpu.CompilerParams(needs_layout_passes=False),
        scratch_types=[
            pltpu.VMEM((CHUNK,), jnp.int32),
            pltpu.VMEM((CHUNK,), jnp.int32),
            pltpu.VMEM((CHUNK,), jnp.int32),
            pltpu.VMEM((CHUNK,), jnp.int32),
            pltpu.VMEM((CHUNK,), jnp.int32),
            pltpu.VMEM((CHUNK,), jnp.int32),
            pltpu.VMEM((128,), jnp.float32),
            pltpu.VMEM((CHUNK, 128), jnp.float32),
            pltpu.VMEM((CHUNK, 128), jnp.float32),
            pltpu.VMEM((CHUNK, 128), jnp.float32),
            pltpu.VMEM_SHARED((NP, 128), jnp.float32),
            pltpu.VMEM_SHARED((NDEN, 128), jnp.float32),
            pltpu.SemaphoreType.DMA,
            pltpu.SemaphoreType.DMA,
        ],
    )
    return f(src, dstg, dstp, dstm, XL, XR, ATT)


def _s3_body(o0_ref, o1_ref, d0_ref, d1_ref, sk_ref, batch_ref, bias_ref,
             g1w_ref, g1b_ref, g1a_ref, g2w_ref, g2b_ref, g2a_ref,
             y_ref, s1m, s2m, s1s, s2s, cntm):
    p = pl.program_id(0)
    o0 = o0_ref[...]
    o1 = o1_ref[...]
    d0 = d0_ref[...]
    d1 = d1_ref[...]
    den = jnp.concatenate(
        [jnp.broadcast_to(d0[:, 0:1], (128, 64)),
         jnp.broadcast_to(d0[:, 1:2], (128, 64)),
         jnp.broadcast_to(d1[:, 0:1], (128, 64)),
         jnp.broadcast_to(d1[:, 1:2], (128, 64))], axis=1)
    den = jnp.maximum(den, 1e-30)
    x_main = jnp.concatenate([o0, o1], axis=1) / den + bias_ref[...]
    x_skip = sk_ref[...]

    gids = lax.broadcasted_iota(jnp.int32, (128, G), 1)
    oh = jnp.where(batch_ref[...] == gids, 1.0, 0.0)

    @pl.when(p == 0)
    def _accum():
        @pl.when(pl.program_id(1) == 0)
        def _init():
            s1m[...] = jnp.zeros_like(s1m)
            s2m[...] = jnp.zeros_like(s2m)
            s1s[...] = jnp.zeros_like(s1s)
            s2s[...] = jnp.zeros_like(s2s)
            cntm[...] = jnp.zeros_like(cntm)
        dn = (((0,), (0,)), ((), ()))
        s1m[...] += lax.dot_general(oh, x_main, dn, preferred_element_type=jnp.float32)
        s2m[...] += lax.dot_general(oh, x_main * x_main, dn, preferred_element_type=jnp.float32)
        s1s[...] += lax.dot_general(oh, x_skip, dn, preferred_element_type=jnp.float32)
        s2s[...] += lax.dot_general(oh, x_skip * x_skip, dn, preferred_element_type=jnp.float32)
        cntm[...] += lax.dot_general(oh, jnp.ones((128, OUT_DIM), jnp.float32), dn,
                                     preferred_element_type=jnp.float32)

    @pl.when(p == 1)
    def _apply():
        cnt = jnp.maximum(cntm[...], 1.0)

        def norm(h, S1, S2, w, b, a):
            mean = S1 / cnt
            var = jnp.maximum(S2 / cnt - (2.0 * a - a * a) * mean * mean, 0.0)
            std = jnp.sqrt(var + EPS)
            gm = jnp.dot(oh, mean, preferred_element_type=jnp.float32)
            gs = jnp.dot(oh, std, preferred_element_type=jnp.float32)
            return w * (h - a * gm) / gs + b

        ym = norm(x_main, s1m[...], s2m[...], g1w_ref[...], g1b_ref[...], g1a_ref[...])
        ys = norm(x_skip, s1s[...], s2s[...], g2w_ref[...], g2b_ref[...], g2a_ref[...])
        z = ym + ys
        y_ref[...] = jnp.where(z > 0, z, jnp.exp(jnp.minimum(z, 0.0)) - 1.0)


def _stage3(OUT, DEN, SK, batchp, bias, g1w, g1b, g1a, g2w, g2b, g2a):
    vec = lambda: pl.BlockSpec((1, OUT_DIM), lambda p, i: (0, 0))
    return pl.pallas_call(
        _s3_body,
        grid=(2, NB),
        in_specs=[
            pl.BlockSpec((128, 128), lambda p, i: (i, 0)),
            pl.BlockSpec((128, 128), lambda p, i: (NB + i, 0)),
            pl.BlockSpec((128, 16), lambda p, i: (i, 0)),
            pl.BlockSpec((128, 16), lambda p, i: (NB + i, 0)),
            pl.BlockSpec((128, OUT_DIM), lambda p, i: (i, 0)),
            pl.BlockSpec((128, 1), lambda p, i: (i, 0)),
            vec(), vec(), vec(), vec(), vec(), vec(), vec(),
        ],
        out_specs=pl.BlockSpec((128, OUT_DIM), lambda p, i: (i, 0)),
        out_shape=jax.ShapeDtypeStruct((NP, OUT_DIM), jnp.float32),
        scratch_shapes=[pltpu.VMEM((G, OUT_DIM), jnp.float32)] * 5,
    )(OUT, OUT, DEN, DEN, SK, batchp,
      bias.reshape(1, -1), g1w.reshape(1, -1), g1b.reshape(1, -1),
      g1a.reshape(1, -1), g2w.reshape(1, -1), g2b.reshape(1, -1),
      g2a.reshape(1, -1))


def kernel(x, edge_index, batch, W_l, b_l, W_r, b_r, att, bias,
           W_skip, b_skip, gn1_w, gn1_b, gn1_a, gn2_w, gn2_b, gn2_a):
    xp = jnp.pad(x, ((0, NP - N), (0, 0)))
    batchp = jnp.pad(batch, (0, NP - N), constant_values=G).reshape(NP, 1)

    loop = jnp.arange(N, dtype=jnp.int32)
    src = jnp.concatenate([edge_index[0], loop])
    dst = jnp.concatenate([edge_index[1], loop])
    srcp = jnp.pad(src, (0, EP - E_TOT))
    dstg = jnp.pad(dst, (0, EP - E_TOT), constant_values=DUMMY)
    dstp = jnp.pad(dst // 8, (0, EP - E_TOT), constant_values=DUMMY // 8)
    dstm = jnp.pad(dst % 8, (0, EP - E_TOT))
    ATT = att.reshape(2, 128)

    XL3, XR3, SK = _stage1(xp, W_l, b_l, W_r, b_r, W_skip, b_skip)
    OUT, DENP = _stage2(srcp, dstg, dstp, dstm,
                        XL3.reshape(2 * NP, 128), XR3.reshape(2 * NP, 128), ATT)
    DEN = DENP.reshape(2, NDEN * 8, 16)[:, :NP, :].reshape(2 * NP, 16)
    y = _stage3(OUT, DEN, SK, batchp, bias,
                gn1_w, gn1_b, gn1_a, gn2_w, gn2_b, gn2_a)
    return y[:N]


# speedup vs baseline: 17.6783x; 1.3276x over previous
import functools
import jax
import jax.numpy as jnp
from jax import lax
from jax.experimental import pallas as pl
from jax.experimental.pallas import tpu as pltpu
from jax.experimental.pallas import tpu_sc as plsc

N = 10000
IN_DIM = 256
OUT_DIM = 256
HEADS = 4
C = OUT_DIM // HEADS
G = 64
EPS = 1e-5

NB = 79
NP = NB * 128
ROWS_PER_TILE = NP // 16
DUMMY = N

DROWS_PER_TILE = 80
NDEN = 16 * DROWS_PER_TILE

E_TOT = 160000 + N
CHUNK = 48
CHUNKS_PER_TILE = 224
EP = 16 * CHUNKS_PER_TILE * CHUNK


def _s1_body(x_ref, wl_ref, bl_ref, wr_ref, br_ref, ws_ref, bs_ref,
             xl_ref, xr_ref, sk_ref):
    xb = x_ref[...]
    xl = jnp.dot(xb, wl_ref[...], preferred_element_type=jnp.float32) + bl_ref[...]
    xr = jnp.dot(xb, wr_ref[...], preferred_element_type=jnp.float32) + br_ref[...]
    xl_ref[0] = xl[:, :128]
    xl_ref[1] = xl[:, 128:]
    xr_ref[0] = xr[:, :128]
    xr_ref[1] = xr[:, 128:]
    sk_ref[...] = jnp.dot(xb, ws_ref[...], preferred_element_type=jnp.float32) + bs_ref[...]


def _stage1(xp, W_l, b_l, W_r, b_r, W_skip, b_skip):
    full = lambda s: pl.BlockSpec(s, lambda i: (0,) * len(s))
    return pl.pallas_call(
        _s1_body,
        grid=(NB,),
        in_specs=[
            pl.BlockSpec((128, IN_DIM), lambda i: (i, 0)),
            full((IN_DIM, OUT_DIM)), full((1, OUT_DIM)),
            full((IN_DIM, OUT_DIM)), full((1, OUT_DIM)),
            full((IN_DIM, OUT_DIM)), full((1, OUT_DIM)),
        ],
        out_specs=[
            pl.BlockSpec((2, 128, 128), lambda i: (0, i, 0)),
            pl.BlockSpec((2, 128, 128), lambda i: (0, i, 0)),
            pl.BlockSpec((128, OUT_DIM), lambda i: (i, 0)),
        ],
        out_shape=[
            jax.ShapeDtypeStruct((2, NP, 128), jnp.float32),
            jax.ShapeDtypeStruct((2, NP, 128), jnp.float32),
            jax.ShapeDtypeStruct((NP, OUT_DIM), jnp.float32),
        ],
    )(xp, W_l, b_l.reshape(1, -1), W_r, b_r.reshape(1, -1),
      W_skip, b_skip.reshape(1, -1))


def _sc_body(idx_hbm, xl_hbm, xr_hbm, att_hbm,
             out_hbm, den_hbm,
             idxv0, idxv1, sgv0, sgv1, dgov0, dgov1, dsuv0, dsuv1,
             dspv0, dspv1, attv,
             xl0, xl1, xr0, xr1, dn0, dn1,
             u_sh, den_sh,
             gx0, gx1, gr0, gr1, su0, su1, sd0, sd1):
    c = lax.axis_index("c")
    s = lax.axis_index("s")
    coff = c * NP
    rbase = s * ROWS_PER_TILE
    dbase = s * DROWS_PER_TILE
    cbase = s * CHUNKS_PER_TILE

    idxv = (idxv0, idxv1)
    sgv = (sgv0, sgv1)
    dgov = (dgov0, dgov1)
    dsuv = (dsuv0, dsuv1)
    dspv = (dspv0, dspv1)
    xlb = (xl0, xl1)
    xrb = (xr0, xr1)
    dnb = (dn0, dn1)
    gx = (gx0, gx1)
    gr = (gr0, gr1)
    su = (su0, su1)
    sd = (sd0, sd1)

    def zrow(e, carry):
        for k in range(8):
            xl0[e, pl.ds(16 * k, 16)] = jnp.zeros((16,), jnp.float32)
            dn0[e, pl.ds(16 * k, 16)] = jnp.zeros((16,), jnp.float32)
        return carry
    lax.fori_loop(0, CHUNK, zrow, 0)

    for q in range(13):
        pltpu.sync_copy(xl0, u_sh.at[pl.ds(rbase + q * CHUNK, CHUNK)])
    rem = ROWS_PER_TILE - 13 * CHUNK
    pltpu.sync_copy(xl0.at[pl.ds(0, rem)],
                    u_sh.at[pl.ds(rbase + 13 * CHUNK, rem)])
    pltpu.sync_copy(dn0, den_sh.at[pl.ds(dbase, CHUNK)])
    pltpu.sync_copy(dn0.at[pl.ds(0, DROWS_PER_TILE - CHUNK)],
                    den_sh.at[pl.ds(dbase + CHUNK, DROWS_PER_TILE - CHUNK)])
    plsc.subcore_barrier()

    pltpu.sync_copy(att_hbm.at[c], attv)

    def load_idx_and_fire(j, b):
        pltpu.sync_copy(idx_hbm.at[pl.ds((cbase + j) * (4 * CHUNK), 4 * CHUNK)],
                        idxv[b])
        for k in range(CHUNK // 16):
            sgv[b][pl.ds(16 * k, 16)] = idxv[b][pl.ds(16 * k, 16)] + coff
            dgov[b][pl.ds(16 * k, 16)] = idxv[b][pl.ds(CHUNK + 16 * k, 16)] + coff
            dsuv[b][pl.ds(16 * k, 16)] = idxv[b][pl.ds(CHUNK + 16 * k, 16)]
            dspv[b][pl.ds(16 * k, 16)] = idxv[b][pl.ds(2 * CHUNK + 16 * k, 16)]
        pltpu.async_copy(xl_hbm.at[sgv[b]], xlb[b], gx[b])
        pltpu.async_copy(xr_hbm.at[dgov[b]], xrb[b], gr[b])

    def wait_scatters(b):
        pltpu.make_async_copy(xlb[b], u_sh.at[dsuv[b]], su[b]).wait()
        pltpu.make_async_copy(dnb[b], den_sh.at[dspv[b]], sd[b]).wait()

    def compute_chunk(b):
        xlr = xlb[b]
        xrr = xrb[b]
        dnr = dnb[b]

        def edge_body(e, carry2):
            pvs = []
            for h in range(2):
                acc = jnp.zeros((16,), jnp.float32)
                for k in range(4):
                    off = h * 64 + 16 * k
                    t = xlr[e, pl.ds(off, 16)] + xrr[e, pl.ds(off, 16)]
                    t = jnp.maximum(t, 0.0) + 0.2 * jnp.minimum(t, 0.0)
                    acc = acc + t * attv[pl.ds(off, 16)]
                pv = jnp.exp(jnp.broadcast_to(jnp.sum(acc), (16,)))
                for k in range(4):
                    off = h * 64 + 16 * k
                    xlr[e, pl.ds(off, 16)] = xlr[e, pl.ds(off, 16)] * pv
                pvs.append(pv)
            lane = lax.iota(jnp.int32, 16)
            pv01 = jnp.where(lane == 0, pvs[0],
                             jnp.where(lane == 1, pvs[1], 0.0))
            mv = plsc.load_gather(
                idxv[b], [jnp.broadcast_to(3 * CHUNK + e, (16,)).astype(jnp.int32)])
            for k in range(8):
                dnr[e, pl.ds(16 * k, 16)] = jnp.where(
                    mv == k, pv01, jnp.zeros((16,), jnp.float32))
            return carry2
        lax.fori_loop(0, CHUNK, edge_body, 0)

    load_idx_and_fire(0, 0)

    def pair_body(jb, carry):
        for b in range(2):
            j = 2 * jb + b
            b1 = 1 - b

            @pl.when(j + 1 < CHUNKS_PER_TILE)
            def _prefetch():
                @pl.when(j >= 1)
                def _drain():
                    wait_scatters(b1)
                load_idx_and_fire(j + 1, b1)

            pltpu.make_async_copy(xl_hbm.at[sgv[b]], xlb[b], gx[b]).wait()
            pltpu.make_async_copy(xr_hbm.at[dgov[b]], xrb[b], gr[b]).wait()
            compute_chunk(b)
            pltpu.async_copy(xlb[b], u_sh.at[dsuv[b]], su[b], add=True)
            pltpu.async_copy(dnb[b], den_sh.at[dspv[b]], sd[b], add=True)
        return carry
    lax.fori_loop(0, CHUNKS_PER_TILE // 2, pair_body, 0)
    wait_scatters(0)
    wait_scatters(1)
    plsc.subcore_barrier()

    for q in range(13):
        pltpu.sync_copy(u_sh.at[pl.ds(rbase + q * CHUNK, CHUNK)], xl0)
        pltpu.sync_copy(xl0, out_hbm.at[pl.ds(coff + rbase + q * CHUNK, CHUNK)])
    rem2 = ROWS_PER_TILE - 13 * CHUNK
    pltpu.sync_copy(u_sh.at[pl.ds(rbase + 13 * CHUNK, rem2)], xl0.at[pl.ds(0, rem2)])
    pltpu.sync_copy(xl0.at[pl.ds(0, rem2)],
                    out_hbm.at[pl.ds(coff + rbase + 13 * CHUNK, rem2)])
    pltpu.sync_copy(den_sh.at[pl.ds(dbase, CHUNK)], dn0)
    pltpu.sync_copy(dn0, den_hbm.at[pl.ds(c * NDEN + dbase, CHUNK)])
    drem = DROWS_PER_TILE - CHUNK
    pltpu.sync_copy(den_sh.at[pl.ds(dbase + CHUNK, drem)], dn0.at[pl.ds(0, drem)])
    pltpu.sync_copy(dn0.at[pl.ds(0, drem)],
                    den_hbm.at[pl.ds(c * NDEN + dbase + CHUNK, drem)])


def _stage2(idx4, XL, XR, ATT):
    mesh = plsc.VectorSubcoreMesh(core_axis_name="c", subcore_axis_name="s")
    ibuf = lambda n: pltpu.VMEM((n,), jnp.int32)
    fbuf = lambda: pltpu.VMEM((CHUNK, 128), jnp.float32)
    f = pl.kernel(
        _sc_body,
        out_type=(
            jax.ShapeDtypeStruct((2 * NP, 128), jnp.float32),
            jax.ShapeDtypeStruct((2 * NDEN, 128), jnp.float32),
        ),
        mesh=mesh,
        compiler_params=pltpu.CompilerParams(needs_layout_passes=False),
        scratch_types=[
            ibuf(4 * CHUNK), ibuf(4 * CHUNK),
            ibuf(CHUNK), ibuf(CHUNK),
            ibuf(CHUNK), ibuf(CHUNK),
            ibuf(CHUNK), ibuf(CHUNK),
            ibuf(CHUNK), ibuf(CHUNK),
            pltpu.VMEM((128,), jnp.float32),
            fbuf(), fbuf(),
            fbuf(), fbuf(),
            fbuf(), fbuf(),
            pltpu.VMEM_SHARED((NP, 128), jnp.float32),
            pltpu.VMEM_SHARED((NDEN, 128), jnp.float32),
        ] + [pltpu.SemaphoreType.DMA] * 8,
    )
    return f(idx4, XL, XR, ATT)


def _s3_body(o0_ref, o1_ref, d0_ref, d1_ref, sk_ref, batch_ref, bias_ref,
             g1w_ref, g1b_ref, g1a_ref, g2w_ref, g2b_ref, g2a_ref,
             y_ref, s1m, s2m, s1s, s2s, cntm):
    p = pl.program_id(0)
    o0 = o0_ref[...]
    o1 = o1_ref[...]
    d0 = d0_ref[...]
    d1 = d1_ref[...]
    den = jnp.concatenate(
        [jnp.broadcast_to(d0[:, 0:1], (128, 64)),
         jnp.broadcast_to(d0[:, 1:2], (128, 64)),
         jnp.broadcast_to(d1[:, 0:1], (128, 64)),
         jnp.broadcast_to(d1[:, 1:2], (128, 64))], axis=1)
    den = jnp.maximum(den, 1e-30)
    x_main = jnp.concatenate([o0, o1], axis=1) / den + bias_ref[...]
    x_skip = sk_ref[...]

    gids = lax.broadcasted_iota(jnp.int32, (128, G), 1)
    oh = jnp.where(batch_ref[...] == gids, 1.0, 0.0)

    @pl.when(p == 0)
    def _accum():
        @pl.when(pl.program_id(1) == 0)
        def _init():
            s1m[...] = jnp.zeros_like(s1m)
            s2m[...] = jnp.zeros_like(s2m)
            s1s[...] = jnp.zeros_like(s1s)
            s2s[...] = jnp.zeros_like(s2s)
            cntm[...] = jnp.zeros_like(cntm)
        dn = (((0,), (0,)), ((), ()))
        s1m[...] += lax.dot_general(oh, x_main, dn, preferred_element_type=jnp.float32)
        s2m[...] += lax.dot_general(oh, x_main * x_main, dn, preferred_element_type=jnp.float32)
        s1s[...] += lax.dot_general(oh, x_skip, dn, preferred_element_type=jnp.float32)
        s2s[...] += lax.dot_general(oh, x_skip * x_skip, dn, preferred_element_type=jnp.float32)
        cntm[...] += lax.dot_general(oh, jnp.ones((128, OUT_DIM), jnp.float32), dn,
                                     preferred_element_type=jnp.float32)

    @pl.when(p == 1)
    def _apply():
        cnt = jnp.maximum(cntm[...], 1.0)

        def norm(h, S1, S2, w, b, a):
            mean = S1 / cnt
            var = jnp.maximum(S2 / cnt - (2.0 * a - a * a) * mean * mean, 0.0)
            std = jnp.sqrt(var + EPS)
            gm = jnp.dot(oh, mean, preferred_element_type=jnp.float32)
            gs = jnp.dot(oh, std, preferred_element_type=jnp.float32)
            return w * (h - a * gm) / gs + b

        ym = norm(x_main, s1m[...], s2m[...], g1w_ref[...], g1b_ref[...], g1a_ref[...])
        ys = norm(x_skip, s1s[...], s2s[...], g2w_ref[...], g2b_ref[...], g2a_ref[...])
        z = ym + ys
        y_ref[...] = jnp.where(z > 0, z, jnp.exp(jnp.minimum(z, 0.0)) - 1.0)


def _stage3(OUT, DEN, SK, batchp, bias, g1w, g1b, g1a, g2w, g2b, g2a):
    vec = lambda: pl.BlockSpec((1, OUT_DIM), lambda p, i: (0, 0))
    return pl.pallas_call(
        _s3_body,
        grid=(2, NB),
        in_specs=[
            pl.BlockSpec((128, 128), lambda p, i: (i, 0)),
            pl.BlockSpec((128, 128), lambda p, i: (NB + i, 0)),
            pl.BlockSpec((128, 16), lambda p, i: (i, 0)),
            pl.BlockSpec((128, 16), lambda p, i: (NB + i, 0)),
            pl.BlockSpec((128, OUT_DIM), lambda p, i: (i, 0)),
            pl.BlockSpec((128, 1), lambda p, i: (i, 0)),
            vec(), vec(), vec(), vec(), vec(), vec(), vec(),
        ],
        out_specs=pl.BlockSpec((128, OUT_DIM), lambda p, i: (i, 0)),
        out_shape=jax.ShapeDtypeStruct((NP, OUT_DIM), jnp.float32),
        scratch_shapes=[pltpu.VMEM((G, OUT_DIM), jnp.float32)] * 5,
    )(OUT, OUT, DEN, DEN, SK, batchp,
      bias.reshape(1, -1), g1w.reshape(1, -1), g1b.reshape(1, -1),
      g1a.reshape(1, -1), g2w.reshape(1, -1), g2b.reshape(1, -1),
      g2a.reshape(1, -1))


def kernel(x, edge_index, batch, W_l, b_l, W_r, b_r, att, bias,
           W_skip, b_skip, gn1_w, gn1_b, gn1_a, gn2_w, gn2_b, gn2_a):
    xp = jnp.pad(x, ((0, NP - N), (0, 0)))
    batchp = jnp.pad(batch, (0, NP - N), constant_values=G).reshape(NP, 1)

    loop = jnp.arange(N, dtype=jnp.int32)
    src = jnp.concatenate([edge_index[0], loop])
    dst = jnp.concatenate([edge_index[1], loop])
    srcp = jnp.pad(src, (0, EP - E_TOT))
    dstg = jnp.pad(dst, (0, EP - E_TOT), constant_values=DUMMY)
    dstp = jnp.pad(dst // 8, (0, EP - E_TOT), constant_values=DUMMY // 8)
    dstm = jnp.pad(dst % 8, (0, EP - E_TOT))
    idx4 = (jnp.stack([srcp, dstg, dstp, dstm], axis=0)
            .reshape(4, EP // CHUNK, CHUNK)
            .transpose(1, 0, 2).reshape(4 * EP))
    ATT = att.reshape(2, 128)

    XL3, XR3, SK = _stage1(xp, W_l, b_l, W_r, b_r, W_skip, b_skip)
    OUT, DENP = _stage2(idx4,
                        XL3.reshape(2 * NP, 128), XR3.reshape(2 * NP, 128), ATT)
    DEN = DENP.reshape(2, NDEN * 8, 16)[:, :NP, :].reshape(2 * NP, 16)
    y = _stage3(OUT, DEN, SK, batchp, bias,
                gn1_w, gn1_b, gn1_a, gn2_w, gn2_b, gn2_a)
    return y[:N]


# speedup vs baseline: 17.7473x; 1.0039x over previous
import functools
import jax
import jax.numpy as jnp
from jax import lax
from jax.experimental import pallas as pl
from jax.experimental.pallas import tpu as pltpu
from jax.experimental.pallas import tpu_sc as plsc

N = 10000
IN_DIM = 256
OUT_DIM = 256
HEADS = 4
C = OUT_DIM // HEADS
G = 64
EPS = 1e-5

NB = 79
NP = NB * 128
ROWS_PER_TILE = NP // 16
DUMMY = N

DROWS_PER_TILE = 80
NDEN = 16 * DROWS_PER_TILE

E_TOT = 160000 + N
CHUNK = 48
CHUNKS_PER_TILE = 224
EP = 16 * CHUNKS_PER_TILE * CHUNK


def _s1_body(x_ref, wl_ref, bl_ref, wr_ref, br_ref, ws_ref, bs_ref,
             xl_ref, xr_ref, sk_ref):
    xb = x_ref[...]
    xl = jnp.dot(xb, wl_ref[...], preferred_element_type=jnp.float32) + bl_ref[...]
    xr = jnp.dot(xb, wr_ref[...], preferred_element_type=jnp.float32) + br_ref[...]
    xl_ref[0] = xl[:, :128]
    xl_ref[1] = xl[:, 128:]
    xr_ref[0] = xr[:, :128]
    xr_ref[1] = xr[:, 128:]
    sk_ref[...] = jnp.dot(xb, ws_ref[...], preferred_element_type=jnp.float32) + bs_ref[...]


def _stage1(xp, W_l, b_l, W_r, b_r, W_skip, b_skip):
    full = lambda s: pl.BlockSpec(s, lambda i: (0,) * len(s))
    return pl.pallas_call(
        _s1_body,
        grid=(NB,),
        in_specs=[
            pl.BlockSpec((128, IN_DIM), lambda i: (i, 0)),
            full((IN_DIM, OUT_DIM)), full((1, OUT_DIM)),
            full((IN_DIM, OUT_DIM)), full((1, OUT_DIM)),
            full((IN_DIM, OUT_DIM)), full((1, OUT_DIM)),
        ],
        out_specs=[
            pl.BlockSpec((2, 128, 128), lambda i: (0, i, 0)),
            pl.BlockSpec((2, 128, 128), lambda i: (0, i, 0)),
            pl.BlockSpec((128, OUT_DIM), lambda i: (i, 0)),
        ],
        out_shape=[
            jax.ShapeDtypeStruct((2, NP, 128), jnp.float32),
            jax.ShapeDtypeStruct((2, NP, 128), jnp.float32),
            jax.ShapeDtypeStruct((NP, OUT_DIM), jnp.float32),
        ],
    )(xp, W_l, b_l.reshape(1, -1), W_r, b_r.reshape(1, -1),
      W_skip, b_skip.reshape(1, -1))


def _sc_body(idx_hbm, xl_hbm, xr_hbm, att_hbm,
             out_hbm, den_hbm,
             idxv0, idxv1, sgv0, sgv1, dgov0, dgov1, dsuv0, dsuv1,
             dspv0, dspv1, attv, posb0, posb1,
             xl0, xl1, xr0, xr1, dn0, dn1,
             u_sh, den_sh,
             gx0, gx1, gr0, gr1, su0, su1, sd0, sd1):
    c = lax.axis_index("c")
    s = lax.axis_index("s")
    coff = c * NP
    rbase = s * ROWS_PER_TILE
    dbase = s * DROWS_PER_TILE
    cbase = s * CHUNKS_PER_TILE

    idxv = (idxv0, idxv1)
    sgv = (sgv0, sgv1)
    dgov = (dgov0, dgov1)
    dsuv = (dsuv0, dsuv1)
    dspv = (dspv0, dspv1)
    posb = (posb0, posb1)
    xlb = (xl0, xl1)
    xrb = (xr0, xr1)
    dnb = (dn0, dn1)
    gx = (gx0, gx1)
    gr = (gr0, gr1)
    su = (su0, su1)
    sd = (sd0, sd1)

    def zrow(e, carry):
        for k in range(8):
            xl0[e, pl.ds(16 * k, 16)] = jnp.zeros((16,), jnp.float32)
            dn0[e, pl.ds(16 * k, 16)] = jnp.zeros((16,), jnp.float32)
            dn1[e, pl.ds(16 * k, 16)] = jnp.zeros((16,), jnp.float32)
        return carry
    lax.fori_loop(0, CHUNK, zrow, 0)
    for k in range(CHUNK // 16):
        posb0[pl.ds(16 * k, 16)] = jnp.zeros((16,), jnp.int32)
        posb1[pl.ds(16 * k, 16)] = jnp.zeros((16,), jnp.int32)

    for q in range(13):
        pltpu.sync_copy(xl0, u_sh.at[pl.ds(rbase + q * CHUNK, CHUNK)])
    rem = ROWS_PER_TILE - 13 * CHUNK
    pltpu.sync_copy(xl0.at[pl.ds(0, rem)],
                    u_sh.at[pl.ds(rbase + 13 * CHUNK, rem)])
    pltpu.sync_copy(dn0, den_sh.at[pl.ds(dbase, CHUNK)])
    pltpu.sync_copy(dn0.at[pl.ds(0, DROWS_PER_TILE - CHUNK)],
                    den_sh.at[pl.ds(dbase + CHUNK, DROWS_PER_TILE - CHUNK)])
    plsc.subcore_barrier()

    pltpu.sync_copy(att_hbm.at[c], attv)

    def load_idx_and_fire(j, b):
        pltpu.sync_copy(idx_hbm.at[pl.ds((cbase + j) * (4 * CHUNK), 4 * CHUNK)],
                        idxv[b])
        for k in range(CHUNK // 16):
            sgv[b][pl.ds(16 * k, 16)] = idxv[b][pl.ds(16 * k, 16)] + coff
            dgov[b][pl.ds(16 * k, 16)] = idxv[b][pl.ds(CHUNK + 16 * k, 16)] + coff
            dsuv[b][pl.ds(16 * k, 16)] = idxv[b][pl.ds(CHUNK + 16 * k, 16)]
            dspv[b][pl.ds(16 * k, 16)] = idxv[b][pl.ds(2 * CHUNK + 16 * k, 16)]
        pltpu.async_copy(xl_hbm.at[sgv[b]], xlb[b], gx[b])
        pltpu.async_copy(xr_hbm.at[dgov[b]], xrb[b], gr[b])

    def wait_scatters(b):
        pltpu.make_async_copy(xlb[b], u_sh.at[dsuv[b]], su[b]).wait()
        pltpu.make_async_copy(dnb[b], den_sh.at[dspv[b]], sd[b]).wait()

    def compute_chunk(b):
        xlr = xlb[b]
        xrr = xrb[b]
        dnr = dnb[b]
        pb = posb[b]
        attvals = [attv[pl.ds(16 * k, 16)] for k in range(8)]
        lane = lax.iota(jnp.int32, 16)
        m01 = lane < 2
        zv = jnp.zeros((16,), jnp.float32)

        def do_edge(e):
            pvs = []
            for h in range(2):
                acc = zv
                for k in range(4):
                    off = h * 64 + 16 * k
                    t = xlr[e, pl.ds(off, 16)] + xrr[e, pl.ds(off, 16)]
                    t = jnp.maximum(t, 0.0) + 0.2 * jnp.minimum(t, 0.0)
                    acc = acc + t * attvals[h * 4 + k]
                pv = jnp.exp(jnp.broadcast_to(jnp.sum(acc), (16,)))
                for k in range(4):
                    off = h * 64 + 16 * k
                    xlr[e, pl.ds(off, 16)] = xlr[e, pl.ds(off, 16)] * pv
                pvs.append(pv)
            pv01 = jnp.where(lane == 0, pvs[0],
                             jnp.where(lane == 1, pvs[1], zv))
            mv = plsc.load_gather(
                idxv[b], [jnp.broadcast_to(3 * CHUNK + e, (16,)).astype(jnp.int32)])
            ev = jnp.broadcast_to(e, (16,)).astype(jnp.int32)
            oldm = plsc.load_gather(pb, [ev])
            plsc.store_scatter(dnr, [ev, oldm * 16 + lane], zv, mask=m01)
            plsc.store_scatter(dnr, [ev, mv * 16 + lane], pv01, mask=m01)
            plsc.store_scatter(pb, [ev], mv, mask=lane == 0)

        def edge_body(i, carry2):
            do_edge(2 * i)
            do_edge(2 * i + 1)
            return carry2
        lax.fori_loop(0, CHUNK // 2, edge_body, 0)

    load_idx_and_fire(0, 0)

    def pair_body(jb, carry):
        for b in range(2):
            j = 2 * jb + b
            b1 = 1 - b

            @pl.when(j + 1 < CHUNKS_PER_TILE)
            def _prefetch():
                @pl.when(j >= 1)
                def _drain():
                    wait_scatters(b1)
                load_idx_and_fire(j + 1, b1)

            pltpu.make_async_copy(xl_hbm.at[sgv[b]], xlb[b], gx[b]).wait()
            pltpu.make_async_copy(xr_hbm.at[dgov[b]], xrb[b], gr[b]).wait()
            compute_chunk(b)
            pltpu.async_copy(xlb[b], u_sh.at[dsuv[b]], su[b], add=True)
            pltpu.async_copy(dnb[b], den_sh.at[dspv[b]], sd[b], add=True)
        return carry
    lax.fori_loop(0, CHUNKS_PER_TILE // 2, pair_body, 0)
    wait_scatters(0)
    wait_scatters(1)
    plsc.subcore_barrier()

    for q in range(13):
        pltpu.sync_copy(u_sh.at[pl.ds(rbase + q * CHUNK, CHUNK)], xl0)
        pltpu.sync_copy(xl0, out_hbm.at[pl.ds(coff + rbase + q * CHUNK, CHUNK)])
    rem2 = ROWS_PER_TILE - 13 * CHUNK
    pltpu.sync_copy(u_sh.at[pl.ds(rbase + 13 * CHUNK, rem2)], xl0.at[pl.ds(0, rem2)])
    pltpu.sync_copy(xl0.at[pl.ds(0, rem2)],
                    out_hbm.at[pl.ds(coff + rbase + 13 * CHUNK, rem2)])
    pltpu.sync_copy(den_sh.at[pl.ds(dbase, CHUNK)], dn0)
    pltpu.sync_copy(dn0, den_hbm.at[pl.ds(c * NDEN + dbase, CHUNK)])
    drem = DROWS_PER_TILE - CHUNK
    pltpu.sync_copy(den_sh.at[pl.ds(dbase + CHUNK, drem)], dn0.at[pl.ds(0, drem)])
    pltpu.sync_copy(dn0.at[pl.ds(0, drem)],
                    den_hbm.at[pl.ds(c * NDEN + dbase + CHUNK, drem)])


def _stage2(idx4, XL, XR, ATT):
    mesh = plsc.VectorSubcoreMesh(core_axis_name="c", subcore_axis_name="s")
    ibuf = lambda n: pltpu.VMEM((n,), jnp.int32)
    fbuf = lambda: pltpu.VMEM((CHUNK, 128), jnp.float32)
    f = pl.kernel(
        _sc_body,
        out_type=(
            jax.ShapeDtypeStruct((2 * NP, 128), jnp.float32),
            jax.ShapeDtypeStruct((2 * NDEN, 128), jnp.float32),
        ),
        mesh=mesh,
        compiler_params=pltpu.CompilerParams(needs_layout_passes=False),
        scratch_types=[
            ibuf(4 * CHUNK), ibuf(4 * CHUNK),
            ibuf(CHUNK), ibuf(CHUNK),
            ibuf(CHUNK), ibuf(CHUNK),
            ibuf(CHUNK), ibuf(CHUNK),
            ibuf(CHUNK), ibuf(CHUNK),
            pltpu.VMEM((128,), jnp.float32),
            ibuf(CHUNK), ibuf(CHUNK),
            fbuf(), fbuf(),
            fbuf(), fbuf(),
            fbuf(), fbuf(),
            pltpu.VMEM_SHARED((NP, 128), jnp.float32),
            pltpu.VMEM_SHARED((NDEN, 128), jnp.float32),
        ] + [pltpu.SemaphoreType.DMA] * 8,
    )
    return f(idx4, XL, XR, ATT)


def _s3_body(o0_ref, o1_ref, d0_ref, d1_ref, sk_ref, batch_ref, bias_ref,
             g1w_ref, g1b_ref, g1a_ref, g2w_ref, g2b_ref, g2a_ref,
             y_ref, s1m, s2m, s1s, s2s, cntm):
    p = pl.program_id(0)
    o0 = o0_ref[...]
    o1 = o1_ref[...]
    d0 = d0_ref[...]
    d1 = d1_ref[...]
    den = jnp.concatenate(
        [jnp.broadcast_to(d0[:, 0:1], (128, 64)),
         jnp.broadcast_to(d0[:, 1:2], (128, 64)),
         jnp.broadcast_to(d1[:, 0:1], (128, 64)),
         jnp.broadcast_to(d1[:, 1:2], (128, 64))], axis=1)
    den = jnp.maximum(den, 1e-30)
    x_main = jnp.concatenate([o0, o1], axis=1) / den + bias_ref[...]
    x_skip = sk_ref[...]

    gids = lax.broadcasted_iota(jnp.int32, (128, G), 1)
    oh = jnp.where(batch_ref[...] == gids, 1.0, 0.0)

    @pl.when(p == 0)
    def _accum():
        @pl.when(pl.program_id(1) == 0)
        def _init():
            s1m[...] = jnp.zeros_like(s1m)
            s2m[...] = jnp.zeros_like(s2m)
            s1s[...] = jnp.zeros_like(s1s)
            s2s[...] = jnp.zeros_like(s2s)
            cntm[...] = jnp.zeros_like(cntm)
        dn = (((0,), (0,)), ((), ()))
        s1m[...] += lax.dot_general(oh, x_main, dn, preferred_element_type=jnp.float32)
        s2m[...] += lax.dot_general(oh, x_main * x_main, dn, preferred_element_type=jnp.float32)
        s1s[...] += lax.dot_general(oh, x_skip, dn, preferred_element_type=jnp.float32)
        s2s[...] += lax.dot_general(oh, x_skip * x_skip, dn, preferred_element_type=jnp.float32)
        cntm[...] += lax.dot_general(oh, jnp.ones((128, OUT_DIM), jnp.float32), dn,
                                     preferred_element_type=jnp.float32)

    @pl.when(p == 1)
    def _apply():
        cnt = jnp.maximum(cntm[...], 1.0)

        def norm(h, S1, S2, w, b, a):
            mean = S1 / cnt
            var = jnp.maximum(S2 / cnt - (2.0 * a - a * a) * mean * mean, 0.0)
            std = jnp.sqrt(var + EPS)
            gm = jnp.dot(oh, mean, preferred_element_type=jnp.float32)
            gs = jnp.dot(oh, std, preferred_element_type=jnp.float32)
            return w * (h - a * gm) / gs + b

        ym = norm(x_main, s1m[...], s2m[...], g1w_ref[...], g1b_ref[...], g1a_ref[...])
        ys = norm(x_skip, s1s[...], s2s[...], g2w_ref[...], g2b_ref[...], g2a_ref[...])
        z = ym + ys
        y_ref[...] = jnp.where(z > 0, z, jnp.exp(jnp.minimum(z, 0.0)) - 1.0)


def _stage3(OUT, DEN, SK, batchp, bias, g1w, g1b, g1a, g2w, g2b, g2a):
    vec = lambda: pl.BlockSpec((1, OUT_DIM), lambda p, i: (0, 0))
    return pl.pallas_call(
        _s3_body,
        grid=(2, NB),
        in_specs=[
            pl.BlockSpec((128, 128), lambda p, i: (i, 0)),
            pl.BlockSpec((128, 128), lambda p, i: (NB + i, 0)),
            pl.BlockSpec((128, 16), lambda p, i: (i, 0)),
            pl.BlockSpec((128, 16), lambda p, i: (NB + i, 0)),
            pl.BlockSpec((128, OUT_DIM), lambda p, i: (i, 0)),
            pl.BlockSpec((128, 1), lambda p, i: (i, 0)),
            vec(), vec(), vec(), vec(), vec(), vec(), vec(),
        ],
        out_specs=pl.BlockSpec((128, OUT_DIM), lambda p, i: (i, 0)),
        out_shape=jax.ShapeDtypeStruct((NP, OUT_DIM), jnp.float32),
        scratch_shapes=[pltpu.VMEM((G, OUT_DIM), jnp.float32)] * 5,
    )(OUT, OUT, DEN, DEN, SK, batchp,
      bias.reshape(1, -1), g1w.reshape(1, -1), g1b.reshape(1, -1),
      g1a.reshape(1, -1), g2w.reshape(1, -1), g2b.reshape(1, -1),
      g2a.reshape(1, -1))


def kernel(x, edge_index, batch, W_l, b_l, W_r, b_r, att, bias,
           W_skip, b_skip, gn1_w, gn1_b, gn1_a, gn2_w, gn2_b, gn2_a):
    xp = jnp.pad(x, ((0, NP - N), (0, 0)))
    batchp = jnp.pad(batch, (0, NP - N), constant_values=G).reshape(NP, 1)

    loop = jnp.arange(N, dtype=jnp.int32)
    src = jnp.concatenate([edge_index[0], loop])
    dst = jnp.concatenate([edge_index[1], loop])
    srcp = jnp.pad(src, (0, EP - E_TOT))
    dstg = jnp.pad(dst, (0, EP - E_TOT), constant_values=DUMMY)
    dstp = jnp.pad(dst // 8, (0, EP - E_TOT), constant_values=DUMMY // 8)
    dstm = jnp.pad(dst % 8, (0, EP - E_TOT))
    idx4 = (jnp.stack([srcp, dstg, dstp, dstm], axis=0)
            .reshape(4, EP // CHUNK, CHUNK)
            .transpose(1, 0, 2).reshape(4 * EP))
    ATT = att.reshape(2, 128)

    XL3, XR3, SK = _stage1(xp, W_l, b_l, W_r, b_r, W_skip, b_skip)
    OUT, DENP = _stage2(idx4,
                        XL3.reshape(2 * NP, 128), XR3.reshape(2 * NP, 128), ATT)
    DEN = DENP.reshape(2, NDEN * 8, 16)[:, :NP, :].reshape(2 * NP, 16)
    y = _stage3(OUT, DEN, SK, batchp, bias,
                gn1_w, gn1_b, gn1_a, gn2_w, gn2_b, gn2_a)
    return y[:N]


# speedup vs baseline: 19.3279x; 1.0891x over previous
import functools
import jax
import jax.numpy as jnp
from jax import lax
from jax.experimental import pallas as pl
from jax.experimental.pallas import tpu as pltpu
from jax.experimental.pallas import tpu_sc as plsc

N = 10000
IN_DIM = 256
OUT_DIM = 256
HEADS = 4
C = OUT_DIM // HEADS
G = 64
EPS = 1e-5

NB = 79
NP = NB * 128
ROWS_PER_TILE = NP // 16
DUMMY = N

DROWS_PER_TILE = 80
NDEN = 16 * DROWS_PER_TILE

E_TOT = 160000 + N
CHUNK = 48
CHUNKS_PER_TILE = 224
EP = 16 * CHUNKS_PER_TILE * CHUNK


def _s1_body(x_ref, wl_ref, bl_ref, wr_ref, br_ref, ws_ref, bs_ref,
             xl_ref, xr_ref, sk_ref):
    xb = x_ref[...]
    xl = jnp.dot(xb, wl_ref[...], preferred_element_type=jnp.float32) + bl_ref[...]
    xr = jnp.dot(xb, wr_ref[...], preferred_element_type=jnp.float32) + br_ref[...]
    xl_ref[0] = xl[:, :128]
    xl_ref[1] = xl[:, 128:]
    xr_ref[0] = xr[:, :128]
    xr_ref[1] = xr[:, 128:]
    sk_ref[...] = jnp.dot(xb, ws_ref[...], preferred_element_type=jnp.float32) + bs_ref[...]


def _stage1(xp, W_l, b_l, W_r, b_r, W_skip, b_skip):
    full = lambda s: pl.BlockSpec(s, lambda i: (0,) * len(s))
    return pl.pallas_call(
        _s1_body,
        grid=(NB,),
        in_specs=[
            pl.BlockSpec((128, IN_DIM), lambda i: (i, 0)),
            full((IN_DIM, OUT_DIM)), full((1, OUT_DIM)),
            full((IN_DIM, OUT_DIM)), full((1, OUT_DIM)),
            full((IN_DIM, OUT_DIM)), full((1, OUT_DIM)),
        ],
        out_specs=[
            pl.BlockSpec((2, 128, 128), lambda i: (0, i, 0)),
            pl.BlockSpec((2, 128, 128), lambda i: (0, i, 0)),
            pl.BlockSpec((128, OUT_DIM), lambda i: (i, 0)),
        ],
        out_shape=[
            jax.ShapeDtypeStruct((2, NP, 128), jnp.float32),
            jax.ShapeDtypeStruct((2, NP, 128), jnp.float32),
            jax.ShapeDtypeStruct((NP, OUT_DIM), jnp.float32),
        ],
    )(xp, W_l, b_l.reshape(1, -1), W_r, b_r.reshape(1, -1),
      W_skip, b_skip.reshape(1, -1))


def _sc_body(idx_hbm, xl_hbm, xr_hbm, att_hbm,
             out_hbm, den_hbm,
             idxv0, idxv1, sgv0, sgv1, dgov0, dgov1, dsuv0, dsuv1,
             dspv0, dspv1, attv, posb0, posb1,
             xl0, xl1, xr0, xr1, dn0, dn1,
             u_sh, den_sh,
             gx0, gx1, gr0, gr1, su0, su1, sd0, sd1, si0, si1):
    c = lax.axis_index("c")
    s = lax.axis_index("s")
    coff = c * NP
    rbase = s * ROWS_PER_TILE
    dbase = s * DROWS_PER_TILE
    cbase = s * CHUNKS_PER_TILE

    idxv = (idxv0, idxv1)
    sgv = (sgv0, sgv1)
    dgov = (dgov0, dgov1)
    dsuv = (dsuv0, dsuv1)
    dspv = (dspv0, dspv1)
    posb = (posb0, posb1)
    xlb = (xl0, xl1)
    xrb = (xr0, xr1)
    dnb = (dn0, dn1)
    gx = (gx0, gx1)
    gr = (gr0, gr1)
    su = (su0, su1)
    sd = (sd0, sd1)
    si = (si0, si1)

    def zrow(e, carry):
        for k in range(8):
            xl0[e, pl.ds(16 * k, 16)] = jnp.zeros((16,), jnp.float32)
            dn0[e, pl.ds(16 * k, 16)] = jnp.zeros((16,), jnp.float32)
            dn1[e, pl.ds(16 * k, 16)] = jnp.zeros((16,), jnp.float32)
        return carry
    lax.fori_loop(0, CHUNK, zrow, 0)
    for k in range(CHUNK // 16):
        posb0[pl.ds(16 * k, 16)] = jnp.zeros((16,), jnp.int32)
        posb1[pl.ds(16 * k, 16)] = jnp.zeros((16,), jnp.int32)

    for q in range(13):
        pltpu.sync_copy(xl0, u_sh.at[pl.ds(rbase + q * CHUNK, CHUNK)])
    rem = ROWS_PER_TILE - 13 * CHUNK
    pltpu.sync_copy(xl0.at[pl.ds(0, rem)],
                    u_sh.at[pl.ds(rbase + 13 * CHUNK, rem)])
    pltpu.sync_copy(dn0, den_sh.at[pl.ds(dbase, CHUNK)])
    pltpu.sync_copy(dn0.at[pl.ds(0, DROWS_PER_TILE - CHUNK)],
                    den_sh.at[pl.ds(dbase + CHUNK, DROWS_PER_TILE - CHUNK)])
    plsc.subcore_barrier()

    pltpu.sync_copy(att_hbm.at[c], attv)

    def idx_slice(j):
        return idx_hbm.at[pl.ds((cbase + j) * (4 * CHUNK), 4 * CHUNK)]

    def build_and_fire(j, b):
        for k in range(CHUNK // 16):
            sgv[b][pl.ds(16 * k, 16)] = idxv[b][pl.ds(16 * k, 16)] + coff
            dgov[b][pl.ds(16 * k, 16)] = idxv[b][pl.ds(CHUNK + 16 * k, 16)] + coff
            dsuv[b][pl.ds(16 * k, 16)] = idxv[b][pl.ds(CHUNK + 16 * k, 16)]
            dspv[b][pl.ds(16 * k, 16)] = idxv[b][pl.ds(2 * CHUNK + 16 * k, 16)]
        pltpu.async_copy(xl_hbm.at[sgv[b]], xlb[b], gx[b])
        pltpu.async_copy(xr_hbm.at[dgov[b]], xrb[b], gr[b])

    def wait_scatters(b):
        pltpu.make_async_copy(xlb[b], u_sh.at[dsuv[b]], su[b]).wait()
        pltpu.make_async_copy(dnb[b], den_sh.at[dspv[b]], sd[b]).wait()

    def compute_chunk(b):
        xlr = xlb[b]
        xrr = xrb[b]
        dnr = dnb[b]
        pb = posb[b]
        attvals = [attv[pl.ds(16 * k, 16)] for k in range(8)]
        lane = lax.iota(jnp.int32, 16)
        m01 = lane < 2
        zv = jnp.zeros((16,), jnp.float32)

        def do_edge(e):
            pvs = []
            for h in range(2):
                acc = zv
                for k in range(4):
                    off = h * 64 + 16 * k
                    t = xlr[e, pl.ds(off, 16)] + xrr[e, pl.ds(off, 16)]
                    t = jnp.maximum(t, 0.0) + 0.2 * jnp.minimum(t, 0.0)
                    acc = acc + t * attvals[h * 4 + k]
                pv = jnp.exp(jnp.broadcast_to(jnp.sum(acc), (16,)))
                for k in range(4):
                    off = h * 64 + 16 * k
                    xlr[e, pl.ds(off, 16)] = xlr[e, pl.ds(off, 16)] * pv
                pvs.append(pv)
            pv01 = jnp.where(lane == 0, pvs[0],
                             jnp.where(lane == 1, pvs[1], zv))
            mv = plsc.load_gather(
                idxv[b], [jnp.broadcast_to(3 * CHUNK + e, (16,)).astype(jnp.int32)])
            ev = jnp.broadcast_to(e, (16,)).astype(jnp.int32)
            oldm = plsc.load_gather(pb, [ev])
            plsc.store_scatter(dnr, [ev, oldm * 16 + lane], zv, mask=m01)
            plsc.store_scatter(dnr, [ev, mv * 16 + lane], pv01, mask=m01)
            plsc.store_scatter(pb, [ev], mv, mask=lane == 0)

        def edge_body(i, carry2):
            do_edge(2 * i)
            do_edge(2 * i + 1)
            return carry2
        lax.fori_loop(0, CHUNK // 2, edge_body, 0)

    pltpu.sync_copy(idx_slice(0), idxv[0])
    build_and_fire(0, 0)
    pltpu.async_copy(idx_slice(1), idxv[1], si[1])

    def pair_body(jb, carry):
        for b in range(2):
            j = 2 * jb + b
            b1 = 1 - b

            @pl.when(j + 1 < CHUNKS_PER_TILE)
            def _prefetch():
                @pl.when(j >= 1)
                def _drain():
                    wait_scatters(b1)
                pltpu.make_async_copy(idx_slice(j + 1), idxv[b1], si[b1]).wait()
                build_and_fire(j + 1, b1)

            pltpu.make_async_copy(xl_hbm.at[sgv[b]], xlb[b], gx[b]).wait()
            pltpu.make_async_copy(xr_hbm.at[dgov[b]], xrb[b], gr[b]).wait()
            compute_chunk(b)
            pltpu.async_copy(xlb[b], u_sh.at[dsuv[b]], su[b], add=True)
            pltpu.async_copy(dnb[b], den_sh.at[dspv[b]], sd[b], add=True)

            @pl.when(j + 2 < CHUNKS_PER_TILE)
            def _idx_prefetch():
                pltpu.async_copy(idx_slice(j + 2), idxv[b], si[b])
        return carry
    lax.fori_loop(0, CHUNKS_PER_TILE // 2, pair_body, 0)
    wait_scatters(0)
    wait_scatters(1)
    plsc.subcore_barrier()

    for q in range(13):
        pltpu.sync_copy(u_sh.at[pl.ds(rbase + q * CHUNK, CHUNK)], xl0)
        pltpu.sync_copy(xl0, out_hbm.at[pl.ds(coff + rbase + q * CHUNK, CHUNK)])
    rem2 = ROWS_PER_TILE - 13 * CHUNK
    pltpu.sync_copy(u_sh.at[pl.ds(rbase + 13 * CHUNK, rem2)], xl0.at[pl.ds(0, rem2)])
    pltpu.sync_copy(xl0.at[pl.ds(0, rem2)],
                    out_hbm.at[pl.ds(coff + rbase + 13 * CHUNK, rem2)])
    pltpu.sync_copy(den_sh.at[pl.ds(dbase, CHUNK)], dn0)
    pltpu.sync_copy(dn0, den_hbm.at[pl.ds(c * NDEN + dbase, CHUNK)])
    drem = DROWS_PER_TILE - CHUNK
    pltpu.sync_copy(den_sh.at[pl.ds(dbase + CHUNK, drem)], dn0.at[pl.ds(0, drem)])
    pltpu.sync_copy(dn0.at[pl.ds(0, drem)],
                    den_hbm.at[pl.ds(c * NDEN + dbase + CHUNK, drem)])


def _stage2(idx4, XL, XR, ATT):
    mesh = plsc.VectorSubcoreMesh(core_axis_name="c", subcore_axis_name="s")
    ibuf = lambda n: pltpu.VMEM((n,), jnp.int32)
    fbuf = lambda: pltpu.VMEM((CHUNK, 128), jnp.float32)
    f = pl.kernel(
        _sc_body,
        out_type=(
            jax.ShapeDtypeStruct((2 * NP, 128), jnp.float32),
            jax.ShapeDtypeStruct((2 * NDEN, 128), jnp.float32),
        ),
        mesh=mesh,
        compiler_params=pltpu.CompilerParams(needs_layout_passes=False),
        scratch_types=[
            ibuf(4 * CHUNK), ibuf(4 * CHUNK),
            ibuf(CHUNK), ibuf(CHUNK),
            ibuf(CHUNK), ibuf(CHUNK),
            ibuf(CHUNK), ibuf(CHUNK),
            ibuf(CHUNK), ibuf(CHUNK),
            pltpu.VMEM((128,), jnp.float32),
            ibuf(CHUNK), ibuf(CHUNK),
            fbuf(), fbuf(),
            fbuf(), fbuf(),
            fbuf(), fbuf(),
            pltpu.VMEM_SHARED((NP, 128), jnp.float32),
            pltpu.VMEM_SHARED((NDEN, 128), jnp.float32),
        ] + [pltpu.SemaphoreType.DMA] * 10,
    )
    return f(idx4, XL, XR, ATT)


def _s3_body(o0_ref, o1_ref, d0_ref, d1_ref, sk_ref, batch_ref, bias_ref,
             g1w_ref, g1b_ref, g1a_ref, g2w_ref, g2b_ref, g2a_ref,
             y_ref, s1m, s2m, s1s, s2s, cntm):
    p = pl.program_id(0)
    o0 = o0_ref[...]
    o1 = o1_ref[...]
    d0 = d0_ref[...]
    d1 = d1_ref[...]
    den = jnp.concatenate(
        [jnp.broadcast_to(d0[:, 0:1], (128, 64)),
         jnp.broadcast_to(d0[:, 1:2], (128, 64)),
         jnp.broadcast_to(d1[:, 0:1], (128, 64)),
         jnp.broadcast_to(d1[:, 1:2], (128, 64))], axis=1)
    den = jnp.maximum(den, 1e-30)
    x_main = jnp.concatenate([o0, o1], axis=1) / den + bias_ref[...]
    x_skip = sk_ref[...]

    gids = lax.broadcasted_iota(jnp.int32, (128, G), 1)
    oh = jnp.where(batch_ref[...] == gids, 1.0, 0.0)

    @pl.when(p == 0)
    def _accum():
        @pl.when(pl.program_id(1) == 0)
        def _init():
            s1m[...] = jnp.zeros_like(s1m)
            s2m[...] = jnp.zeros_like(s2m)
            s1s[...] = jnp.zeros_like(s1s)
            s2s[...] = jnp.zeros_like(s2s)
            cntm[...] = jnp.zeros_like(cntm)
        dn = (((0,), (0,)), ((), ()))
        s1m[...] += lax.dot_general(oh, x_main, dn, preferred_element_type=jnp.float32)
        s2m[...] += lax.dot_general(oh, x_main * x_main, dn, preferred_element_type=jnp.float32)
        s1s[...] += lax.dot_general(oh, x_skip, dn, preferred_element_type=jnp.float32)
        s2s[...] += lax.dot_general(oh, x_skip * x_skip, dn, preferred_element_type=jnp.float32)
        cntm[...] += lax.dot_general(oh, jnp.ones((128, OUT_DIM), jnp.float32), dn,
                                     preferred_element_type=jnp.float32)

    @pl.when(p == 1)
    def _apply():
        cnt = jnp.maximum(cntm[...], 1.0)

        def norm(h, S1, S2, w, b, a):
            mean = S1 / cnt
            var = jnp.maximum(S2 / cnt - (2.0 * a - a * a) * mean * mean, 0.0)
            std = jnp.sqrt(var + EPS)
            gm = jnp.dot(oh, mean, preferred_element_type=jnp.float32)
            gs = jnp.dot(oh, std, preferred_element_type=jnp.float32)
            return w * (h - a * gm) / gs + b

        ym = norm(x_main, s1m[...], s2m[...], g1w_ref[...], g1b_ref[...], g1a_ref[...])
        ys = norm(x_skip, s1s[...], s2s[...], g2w_ref[...], g2b_ref[...], g2a_ref[...])
        z = ym + ys
        y_ref[...] = jnp.where(z > 0, z, jnp.exp(jnp.minimum(z, 0.0)) - 1.0)


def _stage3(OUT, DEN, SK, batchp, bias, g1w, g1b, g1a, g2w, g2b, g2a):
    vec = lambda: pl.BlockSpec((1, OUT_DIM), lambda p, i: (0, 0))
    return pl.pallas_call(
        _s3_body,
        grid=(2, NB),
        in_specs=[
            pl.BlockSpec((128, 128), lambda p, i: (i, 0)),
            pl.BlockSpec((128, 128), lambda p, i: (NB + i, 0)),
            pl.BlockSpec((128, 16), lambda p, i: (i, 0)),
            pl.BlockSpec((128, 16), lambda p, i: (NB + i, 0)),
            pl.BlockSpec((128, OUT_DIM), lambda p, i: (i, 0)),
            pl.BlockSpec((128, 1), lambda p, i: (i, 0)),
            vec(), vec(), vec(), vec(), vec(), vec(), vec(),
        ],
        out_specs=pl.BlockSpec((128, OUT_DIM), lambda p, i: (i, 0)),
        out_shape=jax.ShapeDtypeStruct((NP, OUT_DIM), jnp.float32),
        scratch_shapes=[pltpu.VMEM((G, OUT_DIM), jnp.float32)] * 5,
    )(OUT, OUT, DEN, DEN, SK, batchp,
      bias.reshape(1, -1), g1w.reshape(1, -1), g1b.reshape(1, -1),
      g1a.reshape(1, -1), g2w.reshape(1, -1), g2b.reshape(1, -1),
      g2a.reshape(1, -1))


def kernel(x, edge_index, batch, W_l, b_l, W_r, b_r, att, bias,
           W_skip, b_skip, gn1_w, gn1_b, gn1_a, gn2_w, gn2_b, gn2_a):
    xp = jnp.pad(x, ((0, NP - N), (0, 0)))
    batchp = jnp.pad(batch, (0, NP - N), constant_values=G).reshape(NP, 1)

    loop = jnp.arange(N, dtype=jnp.int32)
    src = jnp.concatenate([edge_index[0], loop])
    dst = jnp.concatenate([edge_index[1], loop])
    srcp = jnp.pad(src, (0, EP - E_TOT))
    dstg = jnp.pad(dst, (0, EP - E_TOT), constant_values=DUMMY)
    dstp = jnp.pad(dst // 8, (0, EP - E_TOT), constant_values=DUMMY // 8)
    dstm = jnp.pad(dst % 8, (0, EP - E_TOT))
    idx4 = (jnp.stack([srcp, dstg, dstp, dstm], axis=0)
            .reshape(4, EP // CHUNK, CHUNK)
            .transpose(1, 0, 2).reshape(4 * EP))
    ATT = att.reshape(2, 128)

    XL3, XR3, SK = _stage1(xp, W_l, b_l, W_r, b_r, W_skip, b_skip)
    OUT, DENP = _stage2(idx4,
                        XL3.reshape(2 * NP, 128), XR3.reshape(2 * NP, 128), ATT)
    DEN = DENP.reshape(2, NDEN * 8, 16)[:, :NP, :].reshape(2 * NP, 16)
    y = _stage3(OUT, DEN, SK, batchp, bias,
                gn1_w, gn1_b, gn1_a, gn2_w, gn2_b, gn2_a)
    return y[:N]


# speedup vs baseline: 38.3943x; 1.9865x over previous
import functools
import jax
import jax.numpy as jnp
from jax import lax
from jax.experimental import pallas as pl
from jax.experimental.pallas import tpu as pltpu
from jax.experimental.pallas import tpu_sc as plsc

N = 10000
IN_DIM = 256
OUT_DIM = 256
HEADS = 4
C = OUT_DIM // HEADS
G = 64
EPS = 1e-5

NB = 79
NP = NB * 128
ROWS_PER_TILE = NP // 16
DUMMY = N

DROWS_PER_TILE = 80
NDEN = 16 * DROWS_PER_TILE

E_TOT = 160000 + N
CHUNK = 48
CHUNKS_PER_TILE = 224
EP = 16 * CHUNKS_PER_TILE * CHUNK


def _s1_body(x_ref, wl_ref, bl_ref, wr_ref, br_ref, ws_ref, bs_ref,
             xl_ref, xr_ref, sk_ref):
    xb = x_ref[...]
    xl = jnp.dot(xb, wl_ref[...], preferred_element_type=jnp.float32) + bl_ref[...]
    xr = jnp.dot(xb, wr_ref[...], preferred_element_type=jnp.float32) + br_ref[...]
    xl_ref[0] = xl[:, :128]
    xl_ref[1] = xl[:, 128:]
    xr_ref[0] = xr[:, :128]
    xr_ref[1] = xr[:, 128:]
    sk_ref[...] = jnp.dot(xb, ws_ref[...], preferred_element_type=jnp.float32) + bs_ref[...]


def _stage1(xp, W_l, b_l, W_r, b_r, W_skip, b_skip):
    full = lambda s: pl.BlockSpec(s, lambda i: (0,) * len(s))
    return pl.pallas_call(
        _s1_body,
        grid=(NB,),
        in_specs=[
            pl.BlockSpec((128, IN_DIM), lambda i: (i, 0)),
            full((IN_DIM, OUT_DIM)), full((1, OUT_DIM)),
            full((IN_DIM, OUT_DIM)), full((1, OUT_DIM)),
            full((IN_DIM, OUT_DIM)), full((1, OUT_DIM)),
        ],
        out_specs=[
            pl.BlockSpec((2, 128, 128), lambda i: (0, i, 0)),
            pl.BlockSpec((2, 128, 128), lambda i: (0, i, 0)),
            pl.BlockSpec((128, OUT_DIM), lambda i: (i, 0)),
        ],
        out_shape=[
            jax.ShapeDtypeStruct((2, NP, 128), jnp.float32),
            jax.ShapeDtypeStruct((2, NP, 128), jnp.float32),
            jax.ShapeDtypeStruct((NP, OUT_DIM), jnp.float32),
        ],
    )(xp, W_l, b_l.reshape(1, -1), W_r, b_r.reshape(1, -1),
      W_skip, b_skip.reshape(1, -1))


def _sc_body(idx_hbm, xl_hbm, xr_hbm, att_hbm,
             out_hbm, den_hbm,
             idxv0, idxv1, sgv0, sgv1, dgov0, dgov1, dsuv0, dsuv1,
             dspv0, dspv1, attv, posb0, posb1,
             xl0, xl1, xr0, xr1, dn0, dn1,
             u_sh, den_sh,
             gx0, gx1, gr0, gr1, su0, su1, sd0, sd1, si0, si1):
    c = lax.axis_index("c")
    s = lax.axis_index("s")
    coff = c * NP
    rbase = s * ROWS_PER_TILE
    dbase = s * DROWS_PER_TILE
    cbase = s * CHUNKS_PER_TILE

    idxv = (idxv0, idxv1)
    sgv = (sgv0, sgv1)
    dgov = (dgov0, dgov1)
    dsuv = (dsuv0, dsuv1)
    dspv = (dspv0, dspv1)
    posb = (posb0, posb1)
    xlb = (xl0, xl1)
    xrb = (xr0, xr1)
    dnb = (dn0, dn1)
    gx = (gx0, gx1)
    gr = (gr0, gr1)
    su = (su0, su1)
    sd = (sd0, sd1)
    si = (si0, si1)

    def zrow(e, carry):
        for k in range(8):
            xl0[e, pl.ds(16 * k, 16)] = jnp.zeros((16,), jnp.float32)
            dn0[e, pl.ds(16 * k, 16)] = jnp.zeros((16,), jnp.float32)
            dn1[e, pl.ds(16 * k, 16)] = jnp.zeros((16,), jnp.float32)
        return carry
    lax.fori_loop(0, CHUNK, zrow, 0)
    for k in range(CHUNK // 16):
        posb0[pl.ds(16 * k, 16)] = jnp.zeros((16,), jnp.int32)
        posb1[pl.ds(16 * k, 16)] = jnp.zeros((16,), jnp.int32)

    for q in range(13):
        pltpu.sync_copy(xl0, u_sh.at[pl.ds(rbase + q * CHUNK, CHUNK)])
    rem = ROWS_PER_TILE - 13 * CHUNK
    pltpu.sync_copy(xl0.at[pl.ds(0, rem)],
                    u_sh.at[pl.ds(rbase + 13 * CHUNK, rem)])
    pltpu.sync_copy(dn0, den_sh.at[pl.ds(dbase, CHUNK)])
    pltpu.sync_copy(dn0.at[pl.ds(0, DROWS_PER_TILE - CHUNK)],
                    den_sh.at[pl.ds(dbase + CHUNK, DROWS_PER_TILE - CHUNK)])
    plsc.subcore_barrier()

    pltpu.sync_copy(att_hbm.at[c], attv)

    def idx_slice(j):
        return idx_hbm.at[pl.ds((cbase + j) * (4 * CHUNK), 4 * CHUNK)]

    def build_and_fire(j, b):
        for k in range(CHUNK // 16):
            sgv[b][pl.ds(16 * k, 16)] = idxv[b][pl.ds(16 * k, 16)] + coff
            dgov[b][pl.ds(16 * k, 16)] = idxv[b][pl.ds(CHUNK + 16 * k, 16)] + coff
            dsuv[b][pl.ds(16 * k, 16)] = idxv[b][pl.ds(CHUNK + 16 * k, 16)]
            dspv[b][pl.ds(16 * k, 16)] = idxv[b][pl.ds(2 * CHUNK + 16 * k, 16)]
        pltpu.async_copy(xl_hbm.at[sgv[b]], xlb[b], gx[b])
        pltpu.async_copy(xr_hbm.at[dgov[b]], xrb[b], gr[b])

    def wait_scatters(b):
        pltpu.make_async_copy(xlb[b], u_sh.at[dsuv[b]], su[b]).wait()
        pltpu.make_async_copy(dnb[b], den_sh.at[dspv[b]], sd[b]).wait()

    def compute_chunk(b):
        xlr = xlb[b]
        xrr = xrb[b]
        dnr = dnb[b]
        pb = posb[b]
        attvals = [attv[pl.ds(16 * k, 16)] for k in range(8)]
        lane = lax.iota(jnp.int32, 16)
        m01 = lane < 2
        zv = jnp.zeros((16,), jnp.float32)

        def do_edge(e):
            pvs = []
            for h in range(2):
                acc = zv
                for k in range(4):
                    off = h * 64 + 16 * k
                    t = xlr[e, pl.ds(off, 16)] + xrr[e, pl.ds(off, 16)]
                    t = jnp.maximum(t, 0.0) + 0.2 * jnp.minimum(t, 0.0)
                    acc = acc + t * attvals[h * 4 + k]
                pv = jnp.exp(jnp.broadcast_to(jnp.sum(acc), (16,)))
                for k in range(4):
                    off = h * 64 + 16 * k
                    xlr[e, pl.ds(off, 16)] = xlr[e, pl.ds(off, 16)] * pv
                pvs.append(pv)
            pv01 = jnp.where(lane == 0, pvs[0],
                             jnp.where(lane == 1, pvs[1], zv))
            mv = plsc.load_gather(
                idxv[b], [jnp.broadcast_to(3 * CHUNK + e, (16,)).astype(jnp.int32)])
            ev = jnp.broadcast_to(e, (16,)).astype(jnp.int32)
            oldm = plsc.load_gather(pb, [ev])
            plsc.store_scatter(dnr, [ev, oldm * 16 + lane], zv, mask=m01)
            plsc.store_scatter(dnr, [ev, mv * 16 + lane], pv01, mask=m01)
            plsc.store_scatter(pb, [ev], mv, mask=lane == 0)

        @functools.partial(plsc.parallel_loop, 0, CHUNK, unroll=4)
        def _edges(e):
            do_edge(e)

    pltpu.sync_copy(idx_slice(0), idxv[0])
    build_and_fire(0, 0)
    pltpu.async_copy(idx_slice(1), idxv[1], si[1])

    def pair_body(jb, carry):
        for b in range(2):
            j = 2 * jb + b
            b1 = 1 - b

            @pl.when(j + 1 < CHUNKS_PER_TILE)
            def _prefetch():
                @pl.when(j >= 1)
                def _drain():
                    wait_scatters(b1)
                pltpu.make_async_copy(idx_slice(j + 1), idxv[b1], si[b1]).wait()
                build_and_fire(j + 1, b1)

            pltpu.make_async_copy(xl_hbm.at[sgv[b]], xlb[b], gx[b]).wait()
            pltpu.make_async_copy(xr_hbm.at[dgov[b]], xrb[b], gr[b]).wait()
            compute_chunk(b)
            pltpu.async_copy(xlb[b], u_sh.at[dsuv[b]], su[b], add=True)
            pltpu.async_copy(dnb[b], den_sh.at[dspv[b]], sd[b], add=True)

            @pl.when(j + 2 < CHUNKS_PER_TILE)
            def _idx_prefetch():
                pltpu.async_copy(idx_slice(j + 2), idxv[b], si[b])
        return carry
    lax.fori_loop(0, CHUNKS_PER_TILE // 2, pair_body, 0)
    wait_scatters(0)
    wait_scatters(1)
    plsc.subcore_barrier()

    for q in range(13):
        pltpu.sync_copy(u_sh.at[pl.ds(rbase + q * CHUNK, CHUNK)], xl0)
        pltpu.sync_copy(xl0, out_hbm.at[pl.ds(coff + rbase + q * CHUNK, CHUNK)])
    rem2 = ROWS_PER_TILE - 13 * CHUNK
    pltpu.sync_copy(u_sh.at[pl.ds(rbase + 13 * CHUNK, rem2)], xl0.at[pl.ds(0, rem2)])
    pltpu.sync_copy(xl0.at[pl.ds(0, rem2)],
                    out_hbm.at[pl.ds(coff + rbase + 13 * CHUNK, rem2)])
    pltpu.sync_copy(den_sh.at[pl.ds(dbase, CHUNK)], dn0)
    pltpu.sync_copy(dn0, den_hbm.at[pl.ds(c * NDEN + dbase, CHUNK)])
    drem = DROWS_PER_TILE - CHUNK
    pltpu.sync_copy(den_sh.at[pl.ds(dbase + CHUNK, drem)], dn0.at[pl.ds(0, drem)])
    pltpu.sync_copy(dn0.at[pl.ds(0, drem)],
                    den_hbm.at[pl.ds(c * NDEN + dbase + CHUNK, drem)])


def _stage2(idx4, XL, XR, ATT):
    mesh = plsc.VectorSubcoreMesh(core_axis_name="c", subcore_axis_name="s")
    ibuf = lambda n: pltpu.VMEM((n,), jnp.int32)
    fbuf = lambda: pltpu.VMEM((CHUNK, 128), jnp.float32)
    f = pl.kernel(
        _sc_body,
        out_type=(
            jax.ShapeDtypeStruct((2 * NP, 128), jnp.float32),
            jax.ShapeDtypeStruct((2 * NDEN, 128), jnp.float32),
        ),
        mesh=mesh,
        compiler_params=pltpu.CompilerParams(needs_layout_passes=False),
        scratch_types=[
            ibuf(4 * CHUNK), ibuf(4 * CHUNK),
            ibuf(CHUNK), ibuf(CHUNK),
            ibuf(CHUNK), ibuf(CHUNK),
            ibuf(CHUNK), ibuf(CHUNK),
            ibuf(CHUNK), ibuf(CHUNK),
            pltpu.VMEM((128,), jnp.float32),
            ibuf(CHUNK), ibuf(CHUNK),
            fbuf(), fbuf(),
            fbuf(), fbuf(),
            fbuf(), fbuf(),
            pltpu.VMEM_SHARED((NP, 128), jnp.float32),
            pltpu.VMEM_SHARED((NDEN, 128), jnp.float32),
        ] + [pltpu.SemaphoreType.DMA] * 10,
    )
    return f(idx4, XL, XR, ATT)


def _s3_body(o0_ref, o1_ref, d0_ref, d1_ref, sk_ref, batch_ref, bias_ref,
             g1w_ref, g1b_ref, g1a_ref, g2w_ref, g2b_ref, g2a_ref,
             y_ref, s1m, s2m, s1s, s2s, cntm):
    p = pl.program_id(0)
    o0 = o0_ref[...]
    o1 = o1_ref[...]
    d0 = d0_ref[...]
    d1 = d1_ref[...]
    den = jnp.concatenate(
        [jnp.broadcast_to(d0[:, 0:1], (128, 64)),
         jnp.broadcast_to(d0[:, 1:2], (128, 64)),
         jnp.broadcast_to(d1[:, 0:1], (128, 64)),
         jnp.broadcast_to(d1[:, 1:2], (128, 64))], axis=1)
    den = jnp.maximum(den, 1e-30)
    x_main = jnp.concatenate([o0, o1], axis=1) / den + bias_ref[...]
    x_skip = sk_ref[...]

    gids = lax.broadcasted_iota(jnp.int32, (128, G), 1)
    oh = jnp.where(batch_ref[...] == gids, 1.0, 0.0)

    @pl.when(p == 0)
    def _accum():
        @pl.when(pl.program_id(1) == 0)
        def _init():
            s1m[...] = jnp.zeros_like(s1m)
            s2m[...] = jnp.zeros_like(s2m)
            s1s[...] = jnp.zeros_like(s1s)
            s2s[...] = jnp.zeros_like(s2s)
            cntm[...] = jnp.zeros_like(cntm)
        dn = (((0,), (0,)), ((), ()))
        s1m[...] += lax.dot_general(oh, x_main, dn, preferred_element_type=jnp.float32)
        s2m[...] += lax.dot_general(oh, x_main * x_main, dn, preferred_element_type=jnp.float32)
        s1s[...] += lax.dot_general(oh, x_skip, dn, preferred_element_type=jnp.float32)
        s2s[...] += lax.dot_general(oh, x_skip * x_skip, dn, preferred_element_type=jnp.float32)
        cntm[...] += lax.dot_general(oh, jnp.ones((128, OUT_DIM), jnp.float32), dn,
                                     preferred_element_type=jnp.float32)

    @pl.when(p == 1)
    def _apply():
        cnt = jnp.maximum(cntm[...], 1.0)

        def norm(h, S1, S2, w, b, a):
            mean = S1 / cnt
            var = jnp.maximum(S2 / cnt - (2.0 * a - a * a) * mean * mean, 0.0)
            std = jnp.sqrt(var + EPS)
            gm = jnp.dot(oh, mean, preferred_element_type=jnp.float32)
            gs = jnp.dot(oh, std, preferred_element_type=jnp.float32)
            return w * (h - a * gm) / gs + b

        ym = norm(x_main, s1m[...], s2m[...], g1w_ref[...], g1b_ref[...], g1a_ref[...])
        ys = norm(x_skip, s1s[...], s2s[...], g2w_ref[...], g2b_ref[...], g2a_ref[...])
        z = ym + ys
        y_ref[...] = jnp.where(z > 0, z, jnp.exp(jnp.minimum(z, 0.0)) - 1.0)


def _stage3(OUT, DEN, SK, batchp, bias, g1w, g1b, g1a, g2w, g2b, g2a):
    vec = lambda: pl.BlockSpec((1, OUT_DIM), lambda p, i: (0, 0))
    return pl.pallas_call(
        _s3_body,
        grid=(2, NB),
        in_specs=[
            pl.BlockSpec((128, 128), lambda p, i: (i, 0)),
            pl.BlockSpec((128, 128), lambda p, i: (NB + i, 0)),
            pl.BlockSpec((128, 16), lambda p, i: (i, 0)),
            pl.BlockSpec((128, 16), lambda p, i: (NB + i, 0)),
            pl.BlockSpec((128, OUT_DIM), lambda p, i: (i, 0)),
            pl.BlockSpec((128, 1), lambda p, i: (i, 0)),
            vec(), vec(), vec(), vec(), vec(), vec(), vec(),
        ],
        out_specs=pl.BlockSpec((128, OUT_DIM), lambda p, i: (i, 0)),
        out_shape=jax.ShapeDtypeStruct((NP, OUT_DIM), jnp.float32),
        scratch_shapes=[pltpu.VMEM((G, OUT_DIM), jnp.float32)] * 5,
    )(OUT, OUT, DEN, DEN, SK, batchp,
      bias.reshape(1, -1), g1w.reshape(1, -1), g1b.reshape(1, -1),
      g1a.reshape(1, -1), g2w.reshape(1, -1), g2b.reshape(1, -1),
      g2a.reshape(1, -1))


def kernel(x, edge_index, batch, W_l, b_l, W_r, b_r, att, bias,
           W_skip, b_skip, gn1_w, gn1_b, gn1_a, gn2_w, gn2_b, gn2_a):
    xp = jnp.pad(x, ((0, NP - N), (0, 0)))
    batchp = jnp.pad(batch, (0, NP - N), constant_values=G).reshape(NP, 1)

    loop = jnp.arange(N, dtype=jnp.int32)
    src = jnp.concatenate([edge_index[0], loop])
    dst = jnp.concatenate([edge_index[1], loop])
    srcp = jnp.pad(src, (0, EP - E_TOT))
    dstg = jnp.pad(dst, (0, EP - E_TOT), constant_values=DUMMY)
    dstp = jnp.pad(dst // 8, (0, EP - E_TOT), constant_values=DUMMY // 8)
    dstm = jnp.pad(dst % 8, (0, EP - E_TOT))
    idx4 = (jnp.stack([srcp, dstg, dstp, dstm], axis=0)
            .reshape(4, EP // CHUNK, CHUNK)
            .transpose(1, 0, 2).reshape(4 * EP))
    ATT = att.reshape(2, 128)

    XL3, XR3, SK = _stage1(xp, W_l, b_l, W_r, b_r, W_skip, b_skip)
    OUT, DENP = _stage2(idx4,
                        XL3.reshape(2 * NP, 128), XR3.reshape(2 * NP, 128), ATT)
    DEN = DENP.reshape(2, NDEN * 8, 16)[:, :NP, :].reshape(2 * NP, 16)
    y = _stage3(OUT, DEN, SK, batchp, bias,
                gn1_w, gn1_b, gn1_a, gn2_w, gn2_b, gn2_a)
    return y[:N]
